# baseline (speedup 1.0000x reference)
import functools

import jax
import jax.numpy as jnp
from jax import lax
from jax.experimental import pallas as pl
from jax.experimental.pallas import tpu as pltpu
from jax.experimental.pallas import tpu_sc as plsc

DM = 128
MS = 8
DMSG = 256
DT = 32
NH = 2
HD = DM // NH

BLK = 512


def _attn_body(mem_ref, mail_ref, dt_ref, wq_ref, bq_ref, wkm_ref, wkt_ref,
               bk_ref, wvm_ref, wvt_ref, bv_ref, wmlp_ref, bmlp_ref,
               lng_ref, lnb_ref, tw_ref, tb_ref, out_ref):
    mem_blk = mem_ref[...]
    mail = mail_ref[...]
    dt = dt_ref[...]

    tf = jnp.cos(dt * tw_ref[...] + tb_ref[...])

    q = mem_blk @ wq_ref[...] + bq_ref[...]
    k2 = mail @ wkm_ref[...] + tf @ wkt_ref[...] + bk_ref[...]
    v2 = mail @ wvm_ref[...] + tf @ wvt_ref[...] + bv_ref[...]

    k3 = k2.reshape(BLK, MS, DM)
    v3 = v2.reshape(BLK, MS, DM)
    q3 = q.reshape(BLK, 1, DM)

    prod = q3 * k3
    lane = lax.broadcasted_iota(jnp.int32, (BLK, MS, DM), 2)
    head0 = lane < HD
    s0 = jnp.sum(jnp.where(head0, prod, 0.0), axis=2)
    s1 = jnp.sum(jnp.where(head0, 0.0, prod), axis=2)

    def _softmax(s):
        s = jnp.where(s >= 0, s, 0.2 * s)
        s = s - jnp.max(s, axis=1, keepdims=True)
        e = jnp.exp(s)
        return e / jnp.sum(e, axis=1, keepdims=True)

    a0 = _softmax(s0)
    a1 = _softmax(s1)
    w3 = jnp.where(head0, a0[:, :, None], a1[:, :, None])
    out = jnp.sum(v3 * w3, axis=1)

    out = out + mem_blk
    mu = jnp.mean(out, axis=1, keepdims=True)
    var = jnp.mean((out - mu) ** 2, axis=1, keepdims=True)
    out = (out - mu) * lax.rsqrt(var + 1e-5) * lng_ref[...] + lnb_ref[...]
    out = jnp.maximum(out @ wmlp_ref[...] + bmlp_ref[...], 0.0)
    out_ref[...] = out


def _attn_stage(mem_g, mail2, dt2, wq_t, b_q, wkm_t, wkt_t, b_k, wvm_t, wvt_t,
                b_v, wmlp_t, b_mlp, ln_g, ln_b, t_w, t_b, total_pad):
    grid = total_pad // BLK
    row_spec = pl.BlockSpec((BLK, DM), lambda i: (i, 0))
    mail_spec = pl.BlockSpec((BLK * MS, DMSG), lambda i: (i, 0))
    dt_spec = pl.BlockSpec((BLK * MS, 1), lambda i: (i, 0))

    def w_spec(shape):
        return pl.BlockSpec(shape, lambda i: (0, 0))

    return pl.pallas_call(
        _attn_body,
        grid=(grid,),
        in_specs=[
            row_spec, mail_spec, dt_spec,
            w_spec((DM, DM)), w_spec((1, DM)),
            w_spec((DMSG, DM)), w_spec((DT, DM)), w_spec((1, DM)),
            w_spec((DMSG, DM)), w_spec((DT, DM)), w_spec((1, DM)),
            w_spec((DM, DM)), w_spec((1, DM)),
            w_spec((1, DM)), w_spec((1, DM)),
            w_spec((1, DT)), w_spec((1, DT)),
        ],
        out_specs=row_spec,
        out_shape=jax.ShapeDtypeStruct((total_pad, DM), jnp.float32),
    )(mem_g, mail2, dt2, wq_t, b_q, wkm_t, wkt_t, b_k, wvm_t, wvt_t, b_v,
      wmlp_t, b_mlp, ln_g, ln_b, t_w, t_b)


def _edge_body(src_ref, dst_ref, neg_ref, ws_ref, bs_ref, wd_ref, bd_ref,
               wo_ref, bo_ref, out_ref):
    hs = src_ref[...] @ ws_ref[...] + bs_ref[...]
    hd = dst_ref[...] @ wd_ref[...] + bd_ref[...]
    hn = neg_ref[...] @ wd_ref[...] + bd_ref[...]
    rp = jnp.maximum(hs + hd, 0.0)
    rn = jnp.maximum(hs + hn, 0.0)
    sp = rp @ wo_ref[...]
    sn = rn @ wo_ref[...]
    out_ref[...] = jnp.concatenate([sp, sn], axis=1) + bo_ref[...]


def _edge_stage(out9k, ws_t, b_src, wd_t, b_dst, wo_t, b_out, size):
    eblk = 600
    grid = size // eblk
    nsb = size // eblk

    return pl.pallas_call(
        _edge_body,
        grid=(grid,),
        in_specs=[
            pl.BlockSpec((eblk, DM), lambda i: (i, 0)),
            pl.BlockSpec((eblk, DM), lambda i: (i + nsb, 0)),
            pl.BlockSpec((eblk, DM), lambda i: (i + 2 * nsb, 0)),
            pl.BlockSpec((DM, DM), lambda i: (0, 0)),
            pl.BlockSpec((1, DM), lambda i: (0, 0)),
            pl.BlockSpec((DM, DM), lambda i: (0, 0)),
            pl.BlockSpec((1, DM), lambda i: (0, 0)),
            pl.BlockSpec((DM, 1), lambda i: (0, 0)),
            pl.BlockSpec((1, 1), lambda i: (0, 0)),
        ],
        out_specs=pl.BlockSpec((eblk, 2), lambda i: (i, 0)),
        out_shape=jax.ShapeDtypeStruct((size, 2), jnp.float32),
    )(out9k, out9k, out9k, ws_t, b_src, wd_t, b_dst, wo_t, b_out)


NSC = 2
NSUB = 16
EPAD = 60160
EPS = EPAD // NSUB
NGRP = EPS // 16
CHUNK = 3840
SHARE = CHUNK // NSUB
TRASH = 128
NCH = 32
ZR2 = 96
WR = 96
NPASS = 7


def _scatter_body(mails_hbm, esrc_hbm, edst_hbm, fsinit_hbm, fdinit_hbm,
                  zero_hbm, zcnt_hbm, ones_hbm, msum_hbm, cnt_hbm,
                  esrc_v, edst_v, fsrc_v, fdst_v, fae_v, fao_v, fge_v, fgo_v,
                  fcn_v, zero_v, zcnt_v, ones_v, rowse_v, rowso_v, cbuf_v,
                  acc_s, cnt_s, sem, sem2):
    c = lax.axis_index("c")
    s = lax.axis_index("s")

    ebase = pl.multiple_of(s * EPS, 16)
    pltpu.sync_copy(esrc_hbm.at[pl.ds(ebase, EPS)], esrc_v)
    pltpu.sync_copy(edst_hbm.at[pl.ds(ebase, EPS)], edst_v)
    pltpu.sync_copy(zero_hbm, zero_v)
    pltpu.sync_copy(zcnt_hbm, zcnt_v)
    pltpu.sync_copy(ones_hbm, ones_v)

    def run_pass(p, acc_s, cnt_s):
        lo = jnp.minimum((NSC * p + c) * CHUNK, 50000 - CHUNK)

        sh = pl.multiple_of(s * SHARE, 8)
        for t in range(SHARE * 2 // ZR2):
            pltpu.sync_copy(zero_v, acc_s.at[pl.ds(2 * sh + t * ZR2, ZR2)])
        pltpu.sync_copy(zcnt_v, cnt_s.at[pl.ds(sh, SHARE)])
        pltpu.sync_copy(fsinit_hbm, fsrc_v)
        pltpu.sync_copy(fdinit_hbm, fdst_v)
        plsc.subcore_barrier()

        def scan_step(g, pos):
            sv = esrc_v[pl.ds(g * 16, 16)]
            dv = edst_v[pl.ds(g * 16, 16)]
            m = jnp.logical_and(sv >= lo, sv < lo + CHUNK)
            csum = plsc.cumsum(jnp.where(m, 1, 0))
            tot = plsc.all_reduce_population_count(m)
            tgt = pos + csum - 1
            row = lax.shift_right_logical(tgt, 7)
            col = jnp.bitwise_and(tgt, 127)
            plsc.store_scatter(fsrc_v, [row, col], sv - lo, mask=m)
            plsc.store_scatter(fdst_v, [row, col], dv, mask=m)
            return pos + tot

        pos = lax.fori_loop(0, NGRP, scan_step,
                            jnp.zeros((16,), jnp.int32), unroll=False)
        nch = (jnp.max(pos) + 127) // 128

        def chunk_step(j, carry):
            for k in range(8):
                d = fdst_v[j, pl.ds(k * 16, 16)]
                a = fsrc_v[j, pl.ds(k * 16, 16)]
                fge_v[pl.ds(k * 16, 16)] = 2 * d
                fgo_v[pl.ds(k * 16, 16)] = 2 * d + 1
                fae_v[pl.ds(k * 16, 16)] = 2 * a
                fao_v[pl.ds(k * 16, 16)] = 2 * a + 1
                fcn_v[pl.ds(k * 16, 16)] = a
            cpe = pltpu.async_copy(mails_hbm.at[fge_v], rowse_v, sem)
            cpo = pltpu.async_copy(mails_hbm.at[fgo_v], rowso_v, sem2)
            cpe.wait()
            pltpu.sync_copy(rowse_v, acc_s.at[fae_v], add=True)
            cpo.wait()
            pltpu.sync_copy(rowso_v, acc_s.at[fao_v], add=True)
            pltpu.sync_copy(ones_v, cnt_s.at[fcn_v], add=True)
            return carry

        lax.fori_loop(0, nch, chunk_step, 0, unroll=False)
        plsc.subcore_barrier()

        for t in range(SHARE * 2 // WR):
            pltpu.sync_copy(acc_s.at[pl.ds(2 * sh + t * WR, WR)],
                            rowse_v.at[pl.ds(0, WR)])
            pltpu.sync_copy(rowse_v.at[pl.ds(0, WR)],
                            msum_hbm.at[pl.ds(2 * (lo + sh) + t * WR, WR)])
        pltpu.sync_copy(cnt_s.at[pl.ds(sh, SHARE)], cbuf_v)
        pltpu.sync_copy(cbuf_v, cnt_hbm.at[pl.ds(lo + sh, SHARE)])
        plsc.subcore_barrier()

    for p in range(NPASS):
        run_pass(p, acc_s, cnt_s)


def _scatter_mean_sc(mails, esrc_p, edst_p):
    fsinit = (CHUNK + jnp.arange(NCH * 128, dtype=jnp.int32) % TRASH
              ).reshape(NCH, 128)
    fdinit = jnp.arange(NCH * 128, dtype=jnp.int32).reshape(NCH, 128)
    zero2d = jnp.zeros((ZR2, 128), jnp.float32)
    zcnt1d = jnp.zeros((SHARE,), jnp.float32)
    ones1d = jnp.ones((128,), jnp.float32)
    mails2 = mails.reshape(-1, 128)

    mesh = plsc.VectorSubcoreMesh(core_axis_name="c", subcore_axis_name="s")
    f = pl.kernel(
        _scatter_body,
        mesh=mesh,
        compiler_params=pltpu.CompilerParams(needs_layout_passes=False),
        out_type=[
            jax.ShapeDtypeStruct((100000, 128), jnp.float32),
            jax.ShapeDtypeStruct((50000,), jnp.float32),
        ],
        scratch_types=[
            pltpu.VMEM((EPS,), jnp.int32),
            pltpu.VMEM((EPS,), jnp.int32),
            pltpu.VMEM((NCH, 128), jnp.int32),
            pltpu.VMEM((NCH, 128), jnp.int32),
            pltpu.VMEM((128,), jnp.int32),
            pltpu.VMEM((128,), jnp.int32),
            pltpu.VMEM((128,), jnp.int32),
            pltpu.VMEM((128,), jnp.int32),
            pltpu.VMEM((128,), jnp.int32),
            pltpu.VMEM((ZR2, 128), jnp.float32),
            pltpu.VMEM((SHARE,), jnp.float32),
            pltpu.VMEM((128,), jnp.float32),
            pltpu.VMEM((128, 128), jnp.float32),
            pltpu.VMEM((128, 128), jnp.float32),
            pltpu.VMEM((SHARE,), jnp.float32),
            pltpu.VMEM_SHARED(((CHUNK + TRASH) * 2, 128), jnp.float32),
            pltpu.VMEM_SHARED((CHUNK + TRASH,), jnp.float32),
            pltpu.SemaphoreType.DMA,
            pltpu.SemaphoreType.DMA,
        ],
    )
    msum2, cnt = f(mails2, esrc_p, edst_p, fsinit, fdinit, zero2d, zcnt1d,
                   ones1d)
    return msum2.reshape(50000, DMSG), cnt


SLOTS = 6144
SPW = SLOTS // (NSC * NSUB)
SCK = 96


def _memscatter_body(out_hbm, nodes_hbm, data_hbm, newmem_ref,
                     nsl_v, dsl_v, gidx_v, sidx_v, grows_v, sem):
    c = lax.axis_index("c")
    s = lax.axis_index("s")
    w = s * NSC + c
    base = pl.multiple_of(w * SPW, 8)

    pltpu.sync_copy(nodes_hbm.at[pl.ds(base, SPW)], nsl_v)
    pltpu.sync_copy(data_hbm.at[pl.ds(base, SPW)], dsl_v)
    for k in range(SPW // SCK):
        for t in range(SCK // 16):
            gidx_v[pl.ds(t * 16, 16)] = dsl_v[pl.ds(k * SCK + t * 16, 16)]
            sidx_v[pl.ds(t * 16, 16)] = nsl_v[pl.ds(k * SCK + t * 16, 16)]
        pltpu.async_copy(out_hbm.at[gidx_v], grows_v, sem).wait()
        pltpu.sync_copy(grows_v, newmem_ref.at[sidx_v])


def _copy_body(in_ref, out_ref):
    out_ref[...] = in_ref[...]


def _mem_update_sc(mem, out9k, nodes6):
    size2 = nodes6.shape[0]
    pos = jnp.arange(size2, dtype=jnp.int32)
    wp = jnp.zeros((mem.shape[0],), jnp.int32).at[nodes6].max(pos)
    dataidx = wp[nodes6]
    nodes_s = jnp.concatenate(
        [nodes6, jnp.full((SLOTS - size2,), nodes6[0], jnp.int32)])
    data_s = jnp.concatenate(
        [dataidx, jnp.full((SLOTS - size2,), dataidx[0], jnp.int32)])

    blk = 2000
    cp = pl.pallas_call(
        _copy_body,
        grid=(mem.shape[0] // blk,),
        in_specs=[pl.BlockSpec((blk, DM), lambda i: (i, 0))],
        out_specs=pl.BlockSpec((blk, DM), lambda i: (i, 0)),
        out_shape=jax.ShapeDtypeStruct(mem.shape, jnp.float32),
    )(mem)

    mesh = plsc.VectorSubcoreMesh(core_axis_name="c", subcore_axis_name="s")
    f = pl.kernel(
        _memscatter_body,
        mesh=mesh,
        compiler_params=pltpu.CompilerParams(needs_layout_passes=False),
        out_type=(),
        scratch_types=[
            pltpu.VMEM((SPW,), jnp.int32),
            pltpu.VMEM((SPW,), jnp.int32),
            pltpu.VMEM((SCK,), jnp.int32),
            pltpu.VMEM((SCK,), jnp.int32),
            pltpu.VMEM((SCK, DM), jnp.float32),
            pltpu.SemaphoreType.DMA,
        ],
    )
    ref = jax.new_ref(cp)
    f(out9k, nodes_s, data_s, ref)
    return ref[...]


GB = 9216
GPW = GB // (NSC * NSUB)
GMC = 16


def _gather_body(mem_hbm, mail_hbm, nodes_hbm, memg_hbm, mailg_hbm,
                 idx_v, memrows_v, mbuf0_v, mbuf1_v, sem0, sem1, sem2):
    c = lax.axis_index("c")
    s = lax.axis_index("s")
    wid = s * NSC + c
    base = pl.multiple_of(wid * GPW, 8)

    pltpu.sync_copy(nodes_hbm.at[pl.ds(base, GPW)], idx_v)
    cpms = [
        pltpu.async_copy(mem_hbm.at[idx_v.at[pl.ds(k * 96, 96)]],
                         memrows_v.at[pl.ds(k * 96, 96)], sem2)
        for k in range(GPW // 96)
    ]

    nmc = GPW // GMC
    bufs = (mbuf0_v, mbuf1_v)
    sems = (sem0, sem1)
    cps = [None, None]
    for t in range(nmc + 1):
        if t < nmc:
            cps[t % 2] = pltpu.async_copy(
                mail_hbm.at[idx_v.at[pl.ds(t * GMC, GMC)]],
                bufs[t % 2], sems[t % 2])
        if t > 0:
            cps[(t - 1) % 2].wait()
            pltpu.sync_copy(bufs[(t - 1) % 2],
                            mailg_hbm.at[pl.ds(base + (t - 1) * GMC, GMC)])

    for cpm in cpms:
        cpm.wait()
    pltpu.sync_copy(memrows_v, memg_hbm.at[pl.ds(base, GPW)])


def _gather_sc(mem, mail2d, nodes_p):
    mesh = plsc.VectorSubcoreMesh(core_axis_name="c", subcore_axis_name="s")
    f = pl.kernel(
        _gather_body,
        mesh=mesh,
        compiler_params=pltpu.CompilerParams(needs_layout_passes=False),
        out_type=[
            jax.ShapeDtypeStruct((GB, DM), jnp.float32),
            jax.ShapeDtypeStruct((GB, MS * DMSG), jnp.float32),
        ],
        scratch_types=[
            pltpu.VMEM((GPW,), jnp.int32),
            pltpu.VMEM((GPW, DM), jnp.float32),
            pltpu.VMEM((GMC, MS * DMSG), jnp.float32),
            pltpu.VMEM((GMC, MS * DMSG), jnp.float32),
            pltpu.SemaphoreType.DMA,
            pltpu.SemaphoreType.DMA,
            pltpu.SemaphoreType.DMA,
        ],
    )
    return f(mem, mail2d, nodes_p)


def _div_body(ms_ref, cnt_ref, out_ref):
    out_ref[...] = ms_ref[...] / jnp.maximum(cnt_ref[...], 1.0)


def _divide_stage(mail_sum, cnt):
    blk = 400
    return pl.pallas_call(
        _div_body,
        grid=(50000 // blk,),
        in_specs=[
            pl.BlockSpec((blk, DMSG), lambda i: (i, 0)),
            pl.BlockSpec((blk, 1), lambda i: (i, 0)),
        ],
        out_specs=pl.BlockSpec((blk, DMSG), lambda i: (i, 0)),
        out_shape=jax.ShapeDtypeStruct((50000, DMSG), jnp.float32),
    )(mail_sum, cnt[:, None])


def kernel(mem, mailbox_mail, mailbox_time, nodes, times, blk_src, blk_dst,
           w_q, b_q, w_k, b_k, w_v, b_v, w_mlp, b_mlp, ln_g, ln_b,
           time_w, time_b, w_src, b_src, w_dst, b_dst, w_out, b_out):
    n = mem.shape[0]
    total = nodes.shape[0]
    size = total // 3
    total_pad = ((total + BLK - 1) // BLK) * BLK

    nodes_p = jnp.concatenate(
        [nodes, jnp.zeros((total_pad - total,), jnp.int32)])
    times_p = jnp.concatenate(
        [times, jnp.zeros((total_pad - total,), jnp.float32)])

    mem_g, mailg = _gather_sc(mem, mailbox_mail.reshape(n, MS * DMSG),
                              nodes_p)
    mail2 = mailg.reshape(total_pad * MS, DMSG)
    mt = mailbox_time[nodes_p]
    dt2 = (times_p[:, None] - mt).reshape(total_pad * MS, 1)

    out = _attn_stage(
        mem_g, mail2, dt2,
        w_q.T, b_q[None, :],
        w_k[:, :DMSG].T, w_k[:, DMSG:].T, b_k[None, :],
        w_v[:, :DMSG].T, w_v[:, DMSG:].T, b_v[None, :],
        w_mlp.T, b_mlp[None, :], ln_g[None, :], ln_b[None, :],
        time_w[None, :], time_b[None, :], total_pad)

    scores = _edge_stage(out, w_src.T, b_src[None, :], w_dst.T,
                         b_dst[None, :], w_out.T, b_out[None, :], size)

    upd = out[:2 * size]
    new_mem = _mem_update_sc(mem, out, nodes[:2 * size])

    m = upd
    src_mail = jnp.concatenate([m[:size], m[size:]], axis=1)
    dst_mail = jnp.concatenate([m[size:], m[:size]], axis=1)
    mails = jnp.concatenate([src_mail, dst_mail], axis=0)

    es = blk_src.shape[0]
    esrc_p = jnp.concatenate(
        [blk_src, jnp.full((EPAD - es,), n, jnp.int32)])
    edst_p = jnp.concatenate(
        [blk_dst, jnp.zeros((EPAD - es,), jnp.int32)])
    mail_sum, cnt = _scatter_mean_sc(mails, esrc_p, edst_p)
    new_mail = _divide_stage(mail_sum, cnt)
    return scores, new_mem, new_mail


# speedup vs baseline: 1.4722x; 1.4722x over previous
import functools

import jax
import jax.numpy as jnp
from jax import lax
from jax.experimental import pallas as pl
from jax.experimental.pallas import tpu as pltpu
from jax.experimental.pallas import tpu_sc as plsc

DM = 128
MS = 8
DMSG = 256
DT = 32
NH = 2
HD = DM // NH

BLK = 512


def _attn_body(mem_ref, mail_ref, dt_ref, wq_ref, bq_ref, wkm_ref, wkt_ref,
               bk_ref, wvm_ref, wvt_ref, bv_ref, wmlp_ref, bmlp_ref,
               lng_ref, lnb_ref, tw_ref, tb_ref, out_ref):
    mem_blk = mem_ref[...]
    mail = mail_ref[...]
    dt = dt_ref[...]

    tf = jnp.cos(dt * tw_ref[...] + tb_ref[...])

    q = mem_blk @ wq_ref[...] + bq_ref[...]
    k2 = mail @ wkm_ref[...] + tf @ wkt_ref[...] + bk_ref[...]
    v2 = mail @ wvm_ref[...] + tf @ wvt_ref[...] + bv_ref[...]

    k3 = k2.reshape(BLK, MS, DM)
    v3 = v2.reshape(BLK, MS, DM)
    q3 = q.reshape(BLK, 1, DM)

    prod = q3 * k3
    lane = lax.broadcasted_iota(jnp.int32, (BLK, MS, DM), 2)
    head0 = lane < HD
    s0 = jnp.sum(jnp.where(head0, prod, 0.0), axis=2)
    s1 = jnp.sum(jnp.where(head0, 0.0, prod), axis=2)

    def _softmax(s):
        s = jnp.where(s >= 0, s, 0.2 * s)
        s = s - jnp.max(s, axis=1, keepdims=True)
        e = jnp.exp(s)
        return e / jnp.sum(e, axis=1, keepdims=True)

    a0 = _softmax(s0)
    a1 = _softmax(s1)
    w3 = jnp.where(head0, a0[:, :, None], a1[:, :, None])
    out = jnp.sum(v3 * w3, axis=1)

    out = out + mem_blk
    mu = jnp.mean(out, axis=1, keepdims=True)
    var = jnp.mean((out - mu) ** 2, axis=1, keepdims=True)
    out = (out - mu) * lax.rsqrt(var + 1e-5) * lng_ref[...] + lnb_ref[...]
    out = jnp.maximum(out @ wmlp_ref[...] + bmlp_ref[...], 0.0)
    out_ref[...] = out


def _attn_stage(mem_g, mail2, dt2, wq_t, b_q, wkm_t, wkt_t, b_k, wvm_t, wvt_t,
                b_v, wmlp_t, b_mlp, ln_g, ln_b, t_w, t_b, total_pad):
    grid = total_pad // BLK
    row_spec = pl.BlockSpec((BLK, DM), lambda i: (i, 0))
    mail_spec = pl.BlockSpec((BLK * MS, DMSG), lambda i: (i, 0))
    dt_spec = pl.BlockSpec((BLK * MS, 1), lambda i: (i, 0))

    def w_spec(shape):
        return pl.BlockSpec(shape, lambda i: (0, 0))

    return pl.pallas_call(
        _attn_body,
        grid=(grid,),
        in_specs=[
            row_spec, mail_spec, dt_spec,
            w_spec((DM, DM)), w_spec((1, DM)),
            w_spec((DMSG, DM)), w_spec((DT, DM)), w_spec((1, DM)),
            w_spec((DMSG, DM)), w_spec((DT, DM)), w_spec((1, DM)),
            w_spec((DM, DM)), w_spec((1, DM)),
            w_spec((1, DM)), w_spec((1, DM)),
            w_spec((1, DT)), w_spec((1, DT)),
        ],
        out_specs=row_spec,
        out_shape=jax.ShapeDtypeStruct((total_pad, DM), jnp.float32),
    )(mem_g, mail2, dt2, wq_t, b_q, wkm_t, wkt_t, b_k, wvm_t, wvt_t, b_v,
      wmlp_t, b_mlp, ln_g, ln_b, t_w, t_b)


def _edge_body(src_ref, dst_ref, neg_ref, ws_ref, bs_ref, wd_ref, bd_ref,
               wo_ref, bo_ref, out_ref):
    hs = src_ref[...] @ ws_ref[...] + bs_ref[...]
    hd = dst_ref[...] @ wd_ref[...] + bd_ref[...]
    hn = neg_ref[...] @ wd_ref[...] + bd_ref[...]
    rp = jnp.maximum(hs + hd, 0.0)
    rn = jnp.maximum(hs + hn, 0.0)
    sp = rp @ wo_ref[...]
    sn = rn @ wo_ref[...]
    out_ref[...] = jnp.concatenate([sp, sn], axis=1) + bo_ref[...]


def _edge_stage(out9k, ws_t, b_src, wd_t, b_dst, wo_t, b_out, size):
    eblk = 600
    grid = size // eblk
    nsb = size // eblk

    return pl.pallas_call(
        _edge_body,
        grid=(grid,),
        in_specs=[
            pl.BlockSpec((eblk, DM), lambda i: (i, 0)),
            pl.BlockSpec((eblk, DM), lambda i: (i + nsb, 0)),
            pl.BlockSpec((eblk, DM), lambda i: (i + 2 * nsb, 0)),
            pl.BlockSpec((DM, DM), lambda i: (0, 0)),
            pl.BlockSpec((1, DM), lambda i: (0, 0)),
            pl.BlockSpec((DM, DM), lambda i: (0, 0)),
            pl.BlockSpec((1, DM), lambda i: (0, 0)),
            pl.BlockSpec((DM, 1), lambda i: (0, 0)),
            pl.BlockSpec((1, 1), lambda i: (0, 0)),
        ],
        out_specs=pl.BlockSpec((eblk, 2), lambda i: (i, 0)),
        out_shape=jax.ShapeDtypeStruct((size, 2), jnp.float32),
    )(out9k, out9k, out9k, ws_t, b_src, wd_t, b_dst, wo_t, b_out)


NSC = 2
NSUB = 16
EPAD = 60160
EPS = EPAD // NSUB
NGRP = EPS // 16
CHUNK = 3840
SHARE = CHUNK // NSUB
TRASH = 128
NCH = 32
ZR2 = 96
WR = 96
NPASS = 7


def _scatter_body(mails_hbm, esrc_hbm, edst_hbm, fsinit_hbm, fdinit_hbm,
                  zero_hbm, zcnt_hbm, ones_hbm, msum_hbm, cnt_hbm,
                  esrc_v, edst_v, fsrc_v, fdst_v, fae_v, fao_v, fge_v, fgo_v,
                  fcn_v, zero_v, zcnt_v, ones_v, rowse_v, rowso_v, cbuf_v,
                  acc_s, cnt_s, sem, sem2):
    c = lax.axis_index("c")
    s = lax.axis_index("s")

    ebase = pl.multiple_of(s * EPS, 16)
    pltpu.sync_copy(esrc_hbm.at[pl.ds(ebase, EPS)], esrc_v)
    pltpu.sync_copy(edst_hbm.at[pl.ds(ebase, EPS)], edst_v)
    pltpu.sync_copy(zero_hbm, zero_v)
    pltpu.sync_copy(zcnt_hbm, zcnt_v)
    pltpu.sync_copy(ones_hbm, ones_v)

    def run_pass(p, acc_s, cnt_s):
        lo = jnp.minimum((NSC * p + c) * CHUNK, 50000 - CHUNK)

        sh = pl.multiple_of(s * SHARE, 8)
        for t in range(SHARE * 2 // ZR2):
            pltpu.sync_copy(zero_v, acc_s.at[pl.ds(2 * sh + t * ZR2, ZR2)])
        pltpu.sync_copy(zcnt_v, cnt_s.at[pl.ds(sh, SHARE)])
        pltpu.sync_copy(fsinit_hbm, fsrc_v)
        pltpu.sync_copy(fdinit_hbm, fdst_v)
        plsc.subcore_barrier()

        def scan_step(g, pos):
            sv = esrc_v[pl.ds(g * 16, 16)]
            dv = edst_v[pl.ds(g * 16, 16)]
            m = jnp.logical_and(sv >= lo, sv < lo + CHUNK)
            csum = plsc.cumsum(jnp.where(m, 1, 0))
            tot = plsc.all_reduce_population_count(m)
            tgt = pos + csum - 1
            row = lax.shift_right_logical(tgt, 7)
            col = jnp.bitwise_and(tgt, 127)
            plsc.store_scatter(fsrc_v, [row, col], sv - lo, mask=m)
            plsc.store_scatter(fdst_v, [row, col], dv, mask=m)
            return pos + tot

        pos = lax.fori_loop(0, NGRP, scan_step,
                            jnp.zeros((16,), jnp.int32), unroll=False)
        nch = (jnp.max(pos) + 127) // 128

        def chunk_step(j, carry):
            for k in range(8):
                d = fdst_v[j, pl.ds(k * 16, 16)]
                a = fsrc_v[j, pl.ds(k * 16, 16)]
                fge_v[pl.ds(k * 16, 16)] = 2 * d
                fgo_v[pl.ds(k * 16, 16)] = 2 * d + 1
                fae_v[pl.ds(k * 16, 16)] = 2 * a
                fao_v[pl.ds(k * 16, 16)] = 2 * a + 1
                fcn_v[pl.ds(k * 16, 16)] = a
            cpe = pltpu.async_copy(mails_hbm.at[fge_v], rowse_v, sem)
            cpo = pltpu.async_copy(mails_hbm.at[fgo_v], rowso_v, sem2)
            cpe.wait()
            pltpu.sync_copy(rowse_v, acc_s.at[fae_v], add=True)
            cpo.wait()
            pltpu.sync_copy(rowso_v, acc_s.at[fao_v], add=True)
            pltpu.sync_copy(ones_v, cnt_s.at[fcn_v], add=True)
            return carry

        lax.fori_loop(0, nch, chunk_step, 0, unroll=False)
        plsc.subcore_barrier()

        for t in range(SHARE * 2 // WR):
            pltpu.sync_copy(acc_s.at[pl.ds(2 * sh + t * WR, WR)],
                            rowse_v.at[pl.ds(0, WR)])
            pltpu.sync_copy(rowse_v.at[pl.ds(0, WR)],
                            msum_hbm.at[pl.ds(2 * (lo + sh) + t * WR, WR)])
        pltpu.sync_copy(cnt_s.at[pl.ds(sh, SHARE)], cbuf_v)
        pltpu.sync_copy(cbuf_v, cnt_hbm.at[pl.ds(lo + sh, SHARE)])
        plsc.subcore_barrier()

    for p in range(NPASS):
        run_pass(p, acc_s, cnt_s)


def _scatter_mean_sc(mails, esrc_p, edst_p):
    fsinit = (CHUNK + jnp.arange(NCH * 128, dtype=jnp.int32) % TRASH
              ).reshape(NCH, 128)
    fdinit = jnp.arange(NCH * 128, dtype=jnp.int32).reshape(NCH, 128)
    zero2d = jnp.zeros((ZR2, 128), jnp.float32)
    zcnt1d = jnp.zeros((SHARE,), jnp.float32)
    ones1d = jnp.ones((128,), jnp.float32)
    mails2 = mails.reshape(-1, 128)

    mesh = plsc.VectorSubcoreMesh(core_axis_name="c", subcore_axis_name="s")
    f = pl.kernel(
        _scatter_body,
        mesh=mesh,
        compiler_params=pltpu.CompilerParams(needs_layout_passes=False),
        out_type=[
            jax.ShapeDtypeStruct((100000, 128), jnp.float32),
            jax.ShapeDtypeStruct((50000,), jnp.float32),
        ],
        scratch_types=[
            pltpu.VMEM((EPS,), jnp.int32),
            pltpu.VMEM((EPS,), jnp.int32),
            pltpu.VMEM((NCH, 128), jnp.int32),
            pltpu.VMEM((NCH, 128), jnp.int32),
            pltpu.VMEM((128,), jnp.int32),
            pltpu.VMEM((128,), jnp.int32),
            pltpu.VMEM((128,), jnp.int32),
            pltpu.VMEM((128,), jnp.int32),
            pltpu.VMEM((128,), jnp.int32),
            pltpu.VMEM((ZR2, 128), jnp.float32),
            pltpu.VMEM((SHARE,), jnp.float32),
            pltpu.VMEM((128,), jnp.float32),
            pltpu.VMEM((128, 128), jnp.float32),
            pltpu.VMEM((128, 128), jnp.float32),
            pltpu.VMEM((SHARE,), jnp.float32),
            pltpu.VMEM_SHARED(((CHUNK + TRASH) * 2, 128), jnp.float32),
            pltpu.VMEM_SHARED((CHUNK + TRASH,), jnp.float32),
            pltpu.SemaphoreType.DMA,
            pltpu.SemaphoreType.DMA,
        ],
    )
    msum2, cnt = f(mails2, esrc_p, edst_p, fsinit, fdinit, zero2d, zcnt1d,
                   ones1d)
    return msum2.reshape(50000, DMSG), cnt


SLOTS = 6144
SPW = SLOTS // (NSC * NSUB)
SCK = 96


def _memscatter_body(out_hbm, nodes_hbm, data_hbm, newmem_ref,
                     nsl_v, dsl_v, gidx_v, sidx_v, grows_v, sem):
    c = lax.axis_index("c")
    s = lax.axis_index("s")
    w = s * NSC + c
    base = pl.multiple_of(w * SPW, 8)

    pltpu.sync_copy(nodes_hbm.at[pl.ds(base, SPW)], nsl_v)
    pltpu.sync_copy(data_hbm.at[pl.ds(base, SPW)], dsl_v)
    for k in range(SPW // SCK):
        for t in range(SCK // 16):
            gidx_v[pl.ds(t * 16, 16)] = dsl_v[pl.ds(k * SCK + t * 16, 16)]
            sidx_v[pl.ds(t * 16, 16)] = nsl_v[pl.ds(k * SCK + t * 16, 16)]
        pltpu.async_copy(out_hbm.at[gidx_v], grows_v, sem).wait()
        pltpu.sync_copy(grows_v, newmem_ref.at[sidx_v])


def _copy_body(in_ref, out_ref):
    out_ref[...] = in_ref[...]


def _mem_update_sc(mem, out9k, nodes6):
    size2 = nodes6.shape[0]
    pos = jnp.arange(size2, dtype=jnp.int32)
    wp = jnp.zeros((mem.shape[0],), jnp.int32).at[nodes6].max(pos)
    dataidx = wp[nodes6]
    nodes_s = jnp.concatenate(
        [nodes6, jnp.full((SLOTS - size2,), nodes6[0], jnp.int32)])
    data_s = jnp.concatenate(
        [dataidx, jnp.full((SLOTS - size2,), dataidx[0], jnp.int32)])

    blk = 2000
    cp = pl.pallas_call(
        _copy_body,
        grid=(mem.shape[0] // blk,),
        in_specs=[pl.BlockSpec((blk, DM), lambda i: (i, 0))],
        out_specs=pl.BlockSpec((blk, DM), lambda i: (i, 0)),
        out_shape=jax.ShapeDtypeStruct(mem.shape, jnp.float32),
    )(mem)

    mesh = plsc.VectorSubcoreMesh(core_axis_name="c", subcore_axis_name="s")
    f = pl.kernel(
        _memscatter_body,
        mesh=mesh,
        compiler_params=pltpu.CompilerParams(needs_layout_passes=False),
        out_type=(),
        scratch_types=[
            pltpu.VMEM((SPW,), jnp.int32),
            pltpu.VMEM((SPW,), jnp.int32),
            pltpu.VMEM((SCK,), jnp.int32),
            pltpu.VMEM((SCK,), jnp.int32),
            pltpu.VMEM((SCK, DM), jnp.float32),
            pltpu.SemaphoreType.DMA,
        ],
    )
    ref = jax.new_ref(cp)
    f(out9k, nodes_s, data_s, ref)
    return ref[...]


GB = 9216
GPW = GB // (NSC * NSUB)
GMC = 16


def _gather_body(mem_hbm, mail_hbm, nodes_hbm, memg_hbm, mailg_hbm,
                 idx_v, memrows_v, mbuf0_v, mbuf1_v, sem0, sem1, sem2):
    c = lax.axis_index("c")
    s = lax.axis_index("s")
    wid = s * NSC + c
    base = pl.multiple_of(wid * GPW, 8)

    pltpu.sync_copy(nodes_hbm.at[pl.ds(base, GPW)], idx_v)
    cpms = [
        pltpu.async_copy(mem_hbm.at[idx_v.at[pl.ds(k * 96, 96)]],
                         memrows_v.at[pl.ds(k * 96, 96)], sem2)
        for k in range(GPW // 96)
    ]

    nmc = GPW // GMC
    bufs = (mbuf0_v, mbuf1_v)
    sems = (sem0, sem1)
    cps = [None, None]
    for t in range(nmc + 1):
        if t < nmc:
            cps[t % 2] = pltpu.async_copy(
                mail_hbm.at[idx_v.at[pl.ds(t * GMC, GMC)]],
                bufs[t % 2], sems[t % 2])
        if t > 0:
            cps[(t - 1) % 2].wait()
            pltpu.sync_copy(bufs[(t - 1) % 2],
                            mailg_hbm.at[pl.ds(base + (t - 1) * GMC, GMC)])

    for cpm in cpms:
        cpm.wait()
    pltpu.sync_copy(memrows_v, memg_hbm.at[pl.ds(base, GPW)])


def _gather_sc(mem, mail2d, nodes_p):
    mesh = plsc.VectorSubcoreMesh(core_axis_name="c", subcore_axis_name="s")
    f = pl.kernel(
        _gather_body,
        mesh=mesh,
        compiler_params=pltpu.CompilerParams(needs_layout_passes=False),
        out_type=[
            jax.ShapeDtypeStruct((GB, DM), jnp.float32),
            jax.ShapeDtypeStruct((GB, MS, DMSG), jnp.float32),
        ],
        scratch_types=[
            pltpu.VMEM((GPW,), jnp.int32),
            pltpu.VMEM((GPW, DM), jnp.float32),
            pltpu.VMEM((GMC, MS, DMSG), jnp.float32),
            pltpu.VMEM((GMC, MS, DMSG), jnp.float32),
            pltpu.SemaphoreType.DMA,
            pltpu.SemaphoreType.DMA,
            pltpu.SemaphoreType.DMA,
        ],
    )
    return f(mem, mail2d, nodes_p)


def _div_body(ms_ref, cnt_ref, out_ref):
    out_ref[...] = ms_ref[...] / jnp.maximum(cnt_ref[...], 1.0)


def _divide_stage(mail_sum, cnt):
    blk = 400
    return pl.pallas_call(
        _div_body,
        grid=(50000 // blk,),
        in_specs=[
            pl.BlockSpec((blk, DMSG), lambda i: (i, 0)),
            pl.BlockSpec((blk, 1), lambda i: (i, 0)),
        ],
        out_specs=pl.BlockSpec((blk, DMSG), lambda i: (i, 0)),
        out_shape=jax.ShapeDtypeStruct((50000, DMSG), jnp.float32),
    )(mail_sum, cnt[:, None])


def kernel(mem, mailbox_mail, mailbox_time, nodes, times, blk_src, blk_dst,
           w_q, b_q, w_k, b_k, w_v, b_v, w_mlp, b_mlp, ln_g, ln_b,
           time_w, time_b, w_src, b_src, w_dst, b_dst, w_out, b_out):
    n = mem.shape[0]
    total = nodes.shape[0]
    size = total // 3
    total_pad = ((total + BLK - 1) // BLK) * BLK

    nodes_p = jnp.concatenate(
        [nodes, jnp.zeros((total_pad - total,), jnp.int32)])
    times_p = jnp.concatenate(
        [times, jnp.zeros((total_pad - total,), jnp.float32)])

    mem_g, mailg = _gather_sc(mem, mailbox_mail, nodes_p)
    mail2 = mailg.reshape(total_pad * MS, DMSG)
    mt = mailbox_time[nodes_p]
    dt2 = (times_p[:, None] - mt).reshape(total_pad * MS, 1)

    out = _attn_stage(
        mem_g, mail2, dt2,
        w_q.T, b_q[None, :],
        w_k[:, :DMSG].T, w_k[:, DMSG:].T, b_k[None, :],
        w_v[:, :DMSG].T, w_v[:, DMSG:].T, b_v[None, :],
        w_mlp.T, b_mlp[None, :], ln_g[None, :], ln_b[None, :],
        time_w[None, :], time_b[None, :], total_pad)

    scores = _edge_stage(out, w_src.T, b_src[None, :], w_dst.T,
                         b_dst[None, :], w_out.T, b_out[None, :], size)

    upd = out[:2 * size]
    new_mem = _mem_update_sc(mem, out, nodes[:2 * size])

    m = upd
    src_mail = jnp.concatenate([m[:size], m[size:]], axis=1)
    dst_mail = jnp.concatenate([m[size:], m[:size]], axis=1)
    mails = jnp.concatenate([src_mail, dst_mail], axis=0)

    es = blk_src.shape[0]
    esrc_p = jnp.concatenate(
        [blk_src, jnp.full((EPAD - es,), n, jnp.int32)])
    edst_p = jnp.concatenate(
        [blk_dst, jnp.zeros((EPAD - es,), jnp.int32)])
    mail_sum, cnt = _scatter_mean_sc(mails, esrc_p, edst_p)
    new_mail = _divide_stage(mail_sum, cnt)
    return scores, new_mem, new_mail


# speedup vs baseline: 1.5069x; 1.0235x over previous
import functools

import jax
import jax.numpy as jnp
from jax import lax
from jax.experimental import pallas as pl
from jax.experimental.pallas import tpu as pltpu
from jax.experimental.pallas import tpu_sc as plsc

DM = 128
MS = 8
DMSG = 256
DT = 32
NH = 2
HD = DM // NH

BLK = 512


def _attn_body(mem_ref, mail_ref, dt_ref, wq_ref, bq_ref, wkm_ref, wkt_ref,
               bk_ref, wvm_ref, wvt_ref, bv_ref, wmlp_ref, bmlp_ref,
               lng_ref, lnb_ref, tw_ref, tb_ref, out_ref):
    mem_blk = mem_ref[...]
    mail = mail_ref[...]
    dt = dt_ref[...]

    tf = jnp.cos(dt * tw_ref[...] + tb_ref[...])

    q = mem_blk @ wq_ref[...] + bq_ref[...]
    k2 = mail @ wkm_ref[...] + tf @ wkt_ref[...] + bk_ref[...]
    v2 = mail @ wvm_ref[...] + tf @ wvt_ref[...] + bv_ref[...]

    k3 = k2.reshape(BLK, MS, DM)
    v3 = v2.reshape(BLK, MS, DM)
    q3 = q.reshape(BLK, 1, DM)

    prod = q3 * k3
    lane = lax.broadcasted_iota(jnp.int32, (BLK, MS, DM), 2)
    head0 = lane < HD
    s0 = jnp.sum(jnp.where(head0, prod, 0.0), axis=2)
    s1 = jnp.sum(jnp.where(head0, 0.0, prod), axis=2)

    def _softmax(s):
        s = jnp.where(s >= 0, s, 0.2 * s)
        s = s - jnp.max(s, axis=1, keepdims=True)
        e = jnp.exp(s)
        return e / jnp.sum(e, axis=1, keepdims=True)

    a0 = _softmax(s0)
    a1 = _softmax(s1)
    w3 = jnp.where(head0, a0[:, :, None], a1[:, :, None])
    out = jnp.sum(v3 * w3, axis=1)

    out = out + mem_blk
    mu = jnp.mean(out, axis=1, keepdims=True)
    var = jnp.mean((out - mu) ** 2, axis=1, keepdims=True)
    out = (out - mu) * lax.rsqrt(var + 1e-5) * lng_ref[...] + lnb_ref[...]
    out = jnp.maximum(out @ wmlp_ref[...] + bmlp_ref[...], 0.0)
    out_ref[...] = out


def _attn_stage(mem_g, mail2, dt2, wq_t, b_q, wkm_t, wkt_t, b_k, wvm_t, wvt_t,
                b_v, wmlp_t, b_mlp, ln_g, ln_b, t_w, t_b, total_pad):
    grid = total_pad // BLK
    row_spec = pl.BlockSpec((BLK, DM), lambda i: (i, 0))
    mail_spec = pl.BlockSpec((BLK * MS, DMSG), lambda i: (i, 0))
    dt_spec = pl.BlockSpec((BLK * MS, 1), lambda i: (i, 0))

    def w_spec(shape):
        return pl.BlockSpec(shape, lambda i: (0, 0))

    return pl.pallas_call(
        _attn_body,
        grid=(grid,),
        in_specs=[
            row_spec, mail_spec, dt_spec,
            w_spec((DM, DM)), w_spec((1, DM)),
            w_spec((DMSG, DM)), w_spec((DT, DM)), w_spec((1, DM)),
            w_spec((DMSG, DM)), w_spec((DT, DM)), w_spec((1, DM)),
            w_spec((DM, DM)), w_spec((1, DM)),
            w_spec((1, DM)), w_spec((1, DM)),
            w_spec((1, DT)), w_spec((1, DT)),
        ],
        out_specs=row_spec,
        out_shape=jax.ShapeDtypeStruct((total_pad, DM), jnp.float32),
    )(mem_g, mail2, dt2, wq_t, b_q, wkm_t, wkt_t, b_k, wvm_t, wvt_t, b_v,
      wmlp_t, b_mlp, ln_g, ln_b, t_w, t_b)


def _edge_body(src_ref, dst_ref, neg_ref, ws_ref, bs_ref, wd_ref, bd_ref,
               wo_ref, bo_ref, out_ref):
    hs = src_ref[...] @ ws_ref[...] + bs_ref[...]
    hd = dst_ref[...] @ wd_ref[...] + bd_ref[...]
    hn = neg_ref[...] @ wd_ref[...] + bd_ref[...]
    rp = jnp.maximum(hs + hd, 0.0)
    rn = jnp.maximum(hs + hn, 0.0)
    sp = rp @ wo_ref[...]
    sn = rn @ wo_ref[...]
    out_ref[...] = jnp.concatenate([sp, sn], axis=1) + bo_ref[...]


def _edge_stage(out9k, ws_t, b_src, wd_t, b_dst, wo_t, b_out, size):
    eblk = 600
    grid = size // eblk
    nsb = size // eblk

    return pl.pallas_call(
        _edge_body,
        grid=(grid,),
        in_specs=[
            pl.BlockSpec((eblk, DM), lambda i: (i, 0)),
            pl.BlockSpec((eblk, DM), lambda i: (i + nsb, 0)),
            pl.BlockSpec((eblk, DM), lambda i: (i + 2 * nsb, 0)),
            pl.BlockSpec((DM, DM), lambda i: (0, 0)),
            pl.BlockSpec((1, DM), lambda i: (0, 0)),
            pl.BlockSpec((DM, DM), lambda i: (0, 0)),
            pl.BlockSpec((1, DM), lambda i: (0, 0)),
            pl.BlockSpec((DM, 1), lambda i: (0, 0)),
            pl.BlockSpec((1, 1), lambda i: (0, 0)),
        ],
        out_specs=pl.BlockSpec((eblk, 2), lambda i: (i, 0)),
        out_shape=jax.ShapeDtypeStruct((size, 2), jnp.float32),
    )(out9k, out9k, out9k, ws_t, b_src, wd_t, b_dst, wo_t, b_out)


NSC = 2
NSUB = 16
EPAD = 60160
EPS = EPAD // NSUB
NGRP = EPS // 16
CHUNK = 3840
SHARE = CHUNK // NSUB
TRASH = 128
NCH = 32
ZR2 = 96
WR = 96
NPASS = 7


def _scatter_body(mails_hbm, esrc_hbm, edst_hbm, fsinit_hbm, fdinit_hbm,
                  zero_hbm, zcnt_hbm, ones_hbm, msum_hbm, cnt_hbm,
                  esrc_v, edst_v, fsrc_v, fdst_v, fae_v, fao_v, fge_v, fgo_v,
                  fcn_v, zero_v, zcnt_v, ones_v, rowse_v, rowso_v, cbuf_v,
                  acc_s, cnt_s, sem, sem2):
    c = lax.axis_index("c")
    s = lax.axis_index("s")

    ebase = pl.multiple_of(s * EPS, 16)
    pltpu.sync_copy(esrc_hbm.at[pl.ds(ebase, EPS)], esrc_v)
    pltpu.sync_copy(edst_hbm.at[pl.ds(ebase, EPS)], edst_v)
    pltpu.sync_copy(zero_hbm, zero_v)
    pltpu.sync_copy(zcnt_hbm, zcnt_v)
    pltpu.sync_copy(ones_hbm, ones_v)

    def run_pass(p, acc_s, cnt_s):
        lo = jnp.minimum((NSC * p + c) * CHUNK, 50000 - CHUNK)

        sh = pl.multiple_of(s * SHARE, 8)
        zcps = [
            pltpu.async_copy(zero_v, acc_s.at[pl.ds(2 * sh + t * ZR2, ZR2)],
                             sem2)
            for t in range(SHARE * 2 // ZR2)
        ]
        zcps.append(pltpu.async_copy(zcnt_v, cnt_s.at[pl.ds(sh, SHARE)],
                                     sem2))
        pltpu.sync_copy(fsinit_hbm, fsrc_v)
        pltpu.sync_copy(fdinit_hbm, fdst_v)

        def scan_step(g, pos):
            sv = esrc_v[pl.ds(g * 16, 16)]
            dv = edst_v[pl.ds(g * 16, 16)]
            m = jnp.logical_and(sv >= lo, sv < lo + CHUNK)
            csum = plsc.cumsum(jnp.where(m, 1, 0))
            tot = plsc.all_reduce_population_count(m)
            tgt = pos + csum - 1
            row = lax.shift_right_logical(tgt, 7)
            col = jnp.bitwise_and(tgt, 127)
            plsc.store_scatter(fsrc_v, [row, col], sv - lo, mask=m)
            plsc.store_scatter(fdst_v, [row, col], dv, mask=m)
            return pos + tot

        pos = lax.fori_loop(0, NGRP, scan_step,
                            jnp.zeros((16,), jnp.int32), unroll=False)
        nch = (jnp.max(pos) + 127) // 128
        for zc in zcps:
            zc.wait()
        plsc.subcore_barrier()

        def chunk_step(j, carry):
            for k in range(8):
                d = fdst_v[j, pl.ds(k * 16, 16)]
                a = fsrc_v[j, pl.ds(k * 16, 16)]
                fge_v[pl.ds(k * 16, 16)] = 2 * d
                fgo_v[pl.ds(k * 16, 16)] = 2 * d + 1
                fae_v[pl.ds(k * 16, 16)] = 2 * a
                fao_v[pl.ds(k * 16, 16)] = 2 * a + 1
                fcn_v[pl.ds(k * 16, 16)] = a
            cpe = pltpu.async_copy(mails_hbm.at[fge_v], rowse_v, sem)
            cpo = pltpu.async_copy(mails_hbm.at[fgo_v], rowso_v, sem2)
            cpe.wait()
            pltpu.sync_copy(rowse_v, acc_s.at[fae_v], add=True)
            cpo.wait()
            pltpu.sync_copy(rowso_v, acc_s.at[fao_v], add=True)
            pltpu.sync_copy(ones_v, cnt_s.at[fcn_v], add=True)
            return carry

        lax.fori_loop(0, nch, chunk_step, 0, unroll=False)
        plsc.subcore_barrier()

        nwr = SHARE * 2 // WR
        bufs = (rowse_v, rowso_v)
        rcp = pltpu.async_copy(acc_s.at[pl.ds(2 * sh, WR)],
                               rowse_v.at[pl.ds(0, WR)], sem)
        for t in range(nwr):
            rcp.wait()
            if t + 1 < nwr:
                rcp = pltpu.async_copy(
                    acc_s.at[pl.ds(2 * sh + (t + 1) * WR, WR)],
                    bufs[(t + 1) % 2].at[pl.ds(0, WR)], sem)
            pltpu.sync_copy(bufs[t % 2].at[pl.ds(0, WR)],
                            msum_hbm.at[pl.ds(2 * (lo + sh) + t * WR, WR)])
        pltpu.sync_copy(cnt_s.at[pl.ds(sh, SHARE)], cbuf_v)
        pltpu.sync_copy(cbuf_v, cnt_hbm.at[pl.ds(lo + sh, SHARE)])
        plsc.subcore_barrier()

    for p in range(NPASS):
        run_pass(p, acc_s, cnt_s)


def _scatter_mean_sc(mails, esrc_p, edst_p):
    fsinit = (CHUNK + jnp.arange(NCH * 128, dtype=jnp.int32) % TRASH
              ).reshape(NCH, 128)
    fdinit = jnp.arange(NCH * 128, dtype=jnp.int32).reshape(NCH, 128)
    zero2d = jnp.zeros((ZR2, 128), jnp.float32)
    zcnt1d = jnp.zeros((SHARE,), jnp.float32)
    ones1d = jnp.ones((128,), jnp.float32)
    mails2 = mails.reshape(-1, 128)

    mesh = plsc.VectorSubcoreMesh(core_axis_name="c", subcore_axis_name="s")
    f = pl.kernel(
        _scatter_body,
        mesh=mesh,
        compiler_params=pltpu.CompilerParams(needs_layout_passes=False),
        out_type=[
            jax.ShapeDtypeStruct((100000, 128), jnp.float32),
            jax.ShapeDtypeStruct((50000,), jnp.float32),
        ],
        scratch_types=[
            pltpu.VMEM((EPS,), jnp.int32),
            pltpu.VMEM((EPS,), jnp.int32),
            pltpu.VMEM((NCH, 128), jnp.int32),
            pltpu.VMEM((NCH, 128), jnp.int32),
            pltpu.VMEM((128,), jnp.int32),
            pltpu.VMEM((128,), jnp.int32),
            pltpu.VMEM((128,), jnp.int32),
            pltpu.VMEM((128,), jnp.int32),
            pltpu.VMEM((128,), jnp.int32),
            pltpu.VMEM((ZR2, 128), jnp.float32),
            pltpu.VMEM((SHARE,), jnp.float32),
            pltpu.VMEM((128,), jnp.float32),
            pltpu.VMEM((128, 128), jnp.float32),
            pltpu.VMEM((128, 128), jnp.float32),
            pltpu.VMEM((SHARE,), jnp.float32),
            pltpu.VMEM_SHARED(((CHUNK + TRASH) * 2, 128), jnp.float32),
            pltpu.VMEM_SHARED((CHUNK + TRASH,), jnp.float32),
            pltpu.SemaphoreType.DMA,
            pltpu.SemaphoreType.DMA,
        ],
    )
    msum2, cnt = f(mails2, esrc_p, edst_p, fsinit, fdinit, zero2d, zcnt1d,
                   ones1d)
    return msum2.reshape(50000, DMSG), cnt


SLOTS = 6144
SPW = SLOTS // (NSC * NSUB)
SCK = 96


def _memscatter_body(out_hbm, nodes_hbm, data_hbm, newmem_ref,
                     nsl_v, dsl_v, gidx_v, sidx_v, grows_v, sem):
    c = lax.axis_index("c")
    s = lax.axis_index("s")
    w = s * NSC + c
    base = pl.multiple_of(w * SPW, 8)

    pltpu.sync_copy(nodes_hbm.at[pl.ds(base, SPW)], nsl_v)
    pltpu.sync_copy(data_hbm.at[pl.ds(base, SPW)], dsl_v)
    for k in range(SPW // SCK):
        for t in range(SCK // 16):
            gidx_v[pl.ds(t * 16, 16)] = dsl_v[pl.ds(k * SCK + t * 16, 16)]
            sidx_v[pl.ds(t * 16, 16)] = nsl_v[pl.ds(k * SCK + t * 16, 16)]
        pltpu.async_copy(out_hbm.at[gidx_v], grows_v, sem).wait()
        pltpu.sync_copy(grows_v, newmem_ref.at[sidx_v])


def _copy_body(in_ref, out_ref):
    out_ref[...] = in_ref[...]


def _mem_update_sc(mem, out9k, nodes6):
    size2 = nodes6.shape[0]
    pos = jnp.arange(size2, dtype=jnp.int32)
    wp = jnp.zeros((mem.shape[0],), jnp.int32).at[nodes6].max(pos)
    dataidx = wp[nodes6]
    nodes_s = jnp.concatenate(
        [nodes6, jnp.full((SLOTS - size2,), nodes6[0], jnp.int32)])
    data_s = jnp.concatenate(
        [dataidx, jnp.full((SLOTS - size2,), dataidx[0], jnp.int32)])

    blk = 2000
    cp = pl.pallas_call(
        _copy_body,
        grid=(mem.shape[0] // blk,),
        in_specs=[pl.BlockSpec((blk, DM), lambda i: (i, 0))],
        out_specs=pl.BlockSpec((blk, DM), lambda i: (i, 0)),
        out_shape=jax.ShapeDtypeStruct(mem.shape, jnp.float32),
    )(mem)

    mesh = plsc.VectorSubcoreMesh(core_axis_name="c", subcore_axis_name="s")
    f = pl.kernel(
        _memscatter_body,
        mesh=mesh,
        compiler_params=pltpu.CompilerParams(needs_layout_passes=False),
        out_type=(),
        scratch_types=[
            pltpu.VMEM((SPW,), jnp.int32),
            pltpu.VMEM((SPW,), jnp.int32),
            pltpu.VMEM((SCK,), jnp.int32),
            pltpu.VMEM((SCK,), jnp.int32),
            pltpu.VMEM((SCK, DM), jnp.float32),
            pltpu.SemaphoreType.DMA,
        ],
    )
    ref = jax.new_ref(cp)
    f(out9k, nodes_s, data_s, ref)
    return ref[...]


GB = 9216
GPW = GB // (NSC * NSUB)
GMC = 16


def _gather_body(mem_hbm, mail_hbm, nodes_hbm, memg_hbm, mailg_hbm,
                 idx_v, memrows_v, mbuf0_v, mbuf1_v, sem0, sem1, sem2):
    c = lax.axis_index("c")
    s = lax.axis_index("s")
    wid = s * NSC + c
    base = pl.multiple_of(wid * GPW, 8)

    pltpu.sync_copy(nodes_hbm.at[pl.ds(base, GPW)], idx_v)
    cpms = [
        pltpu.async_copy(mem_hbm.at[idx_v.at[pl.ds(k * 96, 96)]],
                         memrows_v.at[pl.ds(k * 96, 96)], sem2)
        for k in range(GPW // 96)
    ]

    nmc = GPW // GMC
    bufs = (mbuf0_v, mbuf1_v)
    sems = (sem0, sem1)
    cps = [None, None]
    for t in range(nmc + 1):
        if t < nmc:
            cps[t % 2] = pltpu.async_copy(
                mail_hbm.at[idx_v.at[pl.ds(t * GMC, GMC)]],
                bufs[t % 2], sems[t % 2])
        if t > 0:
            cps[(t - 1) % 2].wait()
            pltpu.sync_copy(bufs[(t - 1) % 2],
                            mailg_hbm.at[pl.ds(base + (t - 1) * GMC, GMC)])

    for cpm in cpms:
        cpm.wait()
    pltpu.sync_copy(memrows_v, memg_hbm.at[pl.ds(base, GPW)])


def _gather_sc(mem, mail2d, nodes_p):
    mesh = plsc.VectorSubcoreMesh(core_axis_name="c", subcore_axis_name="s")
    f = pl.kernel(
        _gather_body,
        mesh=mesh,
        compiler_params=pltpu.CompilerParams(needs_layout_passes=False),
        out_type=[
            jax.ShapeDtypeStruct((GB, DM), jnp.float32),
            jax.ShapeDtypeStruct((GB, MS, DMSG), jnp.float32),
        ],
        scratch_types=[
            pltpu.VMEM((GPW,), jnp.int32),
            pltpu.VMEM((GPW, DM), jnp.float32),
            pltpu.VMEM((GMC, MS, DMSG), jnp.float32),
            pltpu.VMEM((GMC, MS, DMSG), jnp.float32),
            pltpu.SemaphoreType.DMA,
            pltpu.SemaphoreType.DMA,
            pltpu.SemaphoreType.DMA,
        ],
    )
    return f(mem, mail2d, nodes_p)


def _div_body(ms_ref, cnt_ref, out_ref):
    out_ref[...] = ms_ref[...] / jnp.maximum(cnt_ref[...], 1.0)


def _divide_stage(mail_sum, cnt):
    blk = 400
    return pl.pallas_call(
        _div_body,
        grid=(50000 // blk,),
        in_specs=[
            pl.BlockSpec((blk, DMSG), lambda i: (i, 0)),
            pl.BlockSpec((blk, 1), lambda i: (i, 0)),
        ],
        out_specs=pl.BlockSpec((blk, DMSG), lambda i: (i, 0)),
        out_shape=jax.ShapeDtypeStruct((50000, DMSG), jnp.float32),
    )(mail_sum, cnt[:, None])


def kernel(mem, mailbox_mail, mailbox_time, nodes, times, blk_src, blk_dst,
           w_q, b_q, w_k, b_k, w_v, b_v, w_mlp, b_mlp, ln_g, ln_b,
           time_w, time_b, w_src, b_src, w_dst, b_dst, w_out, b_out):
    n = mem.shape[0]
    total = nodes.shape[0]
    size = total // 3
    total_pad = ((total + BLK - 1) // BLK) * BLK

    nodes_p = jnp.concatenate(
        [nodes, jnp.zeros((total_pad - total,), jnp.int32)])
    times_p = jnp.concatenate(
        [times, jnp.zeros((total_pad - total,), jnp.float32)])

    mem_g, mailg = _gather_sc(mem, mailbox_mail, nodes_p)
    mail2 = mailg.reshape(total_pad * MS, DMSG)
    mt = mailbox_time[nodes_p]
    dt2 = (times_p[:, None] - mt).reshape(total_pad * MS, 1)

    out = _attn_stage(
        mem_g, mail2, dt2,
        w_q.T, b_q[None, :],
        w_k[:, :DMSG].T, w_k[:, DMSG:].T, b_k[None, :],
        w_v[:, :DMSG].T, w_v[:, DMSG:].T, b_v[None, :],
        w_mlp.T, b_mlp[None, :], ln_g[None, :], ln_b[None, :],
        time_w[None, :], time_b[None, :], total_pad)

    scores = _edge_stage(out, w_src.T, b_src[None, :], w_dst.T,
                         b_dst[None, :], w_out.T, b_out[None, :], size)

    upd = out[:2 * size]
    new_mem = _mem_update_sc(mem, out, nodes[:2 * size])

    m = upd
    src_mail = jnp.concatenate([m[:size], m[size:]], axis=1)
    dst_mail = jnp.concatenate([m[size:], m[:size]], axis=1)
    mails = jnp.concatenate([src_mail, dst_mail], axis=0)

    es = blk_src.shape[0]
    esrc_p = jnp.concatenate(
        [blk_src, jnp.full((EPAD - es,), n, jnp.int32)])
    edst_p = jnp.concatenate(
        [blk_dst, jnp.zeros((EPAD - es,), jnp.int32)])
    mail_sum, cnt = _scatter_mean_sc(mails, esrc_p, edst_p)
    new_mail = _divide_stage(mail_sum, cnt)
    return scores, new_mem, new_mail


# speedup vs baseline: 1.5260x; 1.0127x over previous
import functools

import jax
import jax.numpy as jnp
from jax import lax
from jax.experimental import pallas as pl
from jax.experimental.pallas import tpu as pltpu
from jax.experimental.pallas import tpu_sc as plsc

DM = 128
MS = 8
DMSG = 256
DT = 32
NH = 2
HD = DM // NH

BLK = 512


def _attn_body(mem_ref, mail_ref, dt_ref, wq_ref, bq_ref, wkm_ref, wkt_ref,
               bk_ref, wvm_ref, wvt_ref, bv_ref, wmlp_ref, bmlp_ref,
               lng_ref, lnb_ref, tw_ref, tb_ref, out_ref):
    mem_blk = mem_ref[...]
    mail = mail_ref[...]
    dt = dt_ref[...]

    tf = jnp.cos(dt * tw_ref[...] + tb_ref[...])

    q = mem_blk @ wq_ref[...] + bq_ref[...]
    k2 = mail @ wkm_ref[...] + tf @ wkt_ref[...] + bk_ref[...]
    v2 = mail @ wvm_ref[...] + tf @ wvt_ref[...] + bv_ref[...]

    k3 = k2.reshape(BLK, MS, DM)
    v3 = v2.reshape(BLK, MS, DM)
    q3 = q.reshape(BLK, 1, DM)

    prod = q3 * k3
    lane = lax.broadcasted_iota(jnp.int32, (BLK, MS, DM), 2)
    head0 = lane < HD
    s0 = jnp.sum(jnp.where(head0, prod, 0.0), axis=2)
    s1 = jnp.sum(jnp.where(head0, 0.0, prod), axis=2)

    def _softmax(s):
        s = jnp.where(s >= 0, s, 0.2 * s)
        s = s - jnp.max(s, axis=1, keepdims=True)
        e = jnp.exp(s)
        return e / jnp.sum(e, axis=1, keepdims=True)

    a0 = _softmax(s0)
    a1 = _softmax(s1)
    w3 = jnp.where(head0, a0[:, :, None], a1[:, :, None])
    out = jnp.sum(v3 * w3, axis=1)

    out = out + mem_blk
    mu = jnp.mean(out, axis=1, keepdims=True)
    var = jnp.mean((out - mu) ** 2, axis=1, keepdims=True)
    out = (out - mu) * lax.rsqrt(var + 1e-5) * lng_ref[...] + lnb_ref[...]
    out = jnp.maximum(out @ wmlp_ref[...] + bmlp_ref[...], 0.0)
    out_ref[...] = out


def _attn_stage(mem_g, mail2, dt2, wq_t, b_q, wkm_t, wkt_t, b_k, wvm_t, wvt_t,
                b_v, wmlp_t, b_mlp, ln_g, ln_b, t_w, t_b, total_pad):
    grid = total_pad // BLK
    row_spec = pl.BlockSpec((BLK, DM), lambda i: (i, 0))
    mail_spec = pl.BlockSpec((BLK * MS, DMSG), lambda i: (i, 0))
    dt_spec = pl.BlockSpec((BLK * MS, 1), lambda i: (i, 0))

    def w_spec(shape):
        return pl.BlockSpec(shape, lambda i: (0, 0))

    return pl.pallas_call(
        _attn_body,
        grid=(grid,),
        in_specs=[
            row_spec, mail_spec, dt_spec,
            w_spec((DM, DM)), w_spec((1, DM)),
            w_spec((DMSG, DM)), w_spec((DT, DM)), w_spec((1, DM)),
            w_spec((DMSG, DM)), w_spec((DT, DM)), w_spec((1, DM)),
            w_spec((DM, DM)), w_spec((1, DM)),
            w_spec((1, DM)), w_spec((1, DM)),
            w_spec((1, DT)), w_spec((1, DT)),
        ],
        out_specs=row_spec,
        out_shape=jax.ShapeDtypeStruct((total_pad, DM), jnp.float32),
    )(mem_g, mail2, dt2, wq_t, b_q, wkm_t, wkt_t, b_k, wvm_t, wvt_t, b_v,
      wmlp_t, b_mlp, ln_g, ln_b, t_w, t_b)


def _edge_body(src_ref, dst_ref, neg_ref, ws_ref, bs_ref, wd_ref, bd_ref,
               wo_ref, bo_ref, out_ref):
    hs = src_ref[...] @ ws_ref[...] + bs_ref[...]
    hd = dst_ref[...] @ wd_ref[...] + bd_ref[...]
    hn = neg_ref[...] @ wd_ref[...] + bd_ref[...]
    rp = jnp.maximum(hs + hd, 0.0)
    rn = jnp.maximum(hs + hn, 0.0)
    sp = rp @ wo_ref[...]
    sn = rn @ wo_ref[...]
    out_ref[...] = jnp.concatenate([sp, sn], axis=1) + bo_ref[...]


def _edge_stage(out9k, ws_t, b_src, wd_t, b_dst, wo_t, b_out, size):
    eblk = 600
    grid = size // eblk
    nsb = size // eblk

    return pl.pallas_call(
        _edge_body,
        grid=(grid,),
        in_specs=[
            pl.BlockSpec((eblk, DM), lambda i: (i, 0)),
            pl.BlockSpec((eblk, DM), lambda i: (i + nsb, 0)),
            pl.BlockSpec((eblk, DM), lambda i: (i + 2 * nsb, 0)),
            pl.BlockSpec((DM, DM), lambda i: (0, 0)),
            pl.BlockSpec((1, DM), lambda i: (0, 0)),
            pl.BlockSpec((DM, DM), lambda i: (0, 0)),
            pl.BlockSpec((1, DM), lambda i: (0, 0)),
            pl.BlockSpec((DM, 1), lambda i: (0, 0)),
            pl.BlockSpec((1, 1), lambda i: (0, 0)),
        ],
        out_specs=pl.BlockSpec((eblk, 2), lambda i: (i, 0)),
        out_shape=jax.ShapeDtypeStruct((size, 2), jnp.float32),
    )(out9k, out9k, out9k, ws_t, b_src, wd_t, b_dst, wo_t, b_out)


NSC = 2
NSUB = 16
EPAD = 60160
EPS = EPAD // NSUB
NGRP = EPS // 16
CHUNK = 5248
SHARE = CHUNK // NSUB
TRASH = 128
NCH = 32
ZR2 = 48
WR = 64
NPASS = 5


def _scatter_body(mails_hbm, esrc_hbm, edst_hbm, fsinit_hbm, fdinit_hbm,
                  zero_hbm, zcnt_hbm, ones_hbm, msum_hbm, cnt_hbm,
                  esrc_v, edst_v, fsrc_v, fdst_v, fae_v, fao_v, fge_v, fgo_v,
                  fcn_v, zero_v, zcnt_v, ones_v, rowse_v, rowso_v, cbuf_v,
                  acc_s, cnt_s, sem, sem2):
    c = lax.axis_index("c")
    s = lax.axis_index("s")

    ebase = pl.multiple_of(s * EPS, 16)
    pltpu.sync_copy(esrc_hbm.at[pl.ds(ebase, EPS)], esrc_v)
    pltpu.sync_copy(edst_hbm.at[pl.ds(ebase, EPS)], edst_v)
    pltpu.sync_copy(zero_hbm, zero_v)
    pltpu.sync_copy(zcnt_hbm, zcnt_v)
    pltpu.sync_copy(ones_hbm, ones_v)

    def run_pass(p, acc_s, cnt_s):
        lo = jnp.minimum((NSC * p + c) * CHUNK, 50000 - CHUNK)

        sh = pl.multiple_of(s * SHARE, 8)
        zcps = [
            pltpu.async_copy(
                zero_v,
                acc_s.at[pl.ds(2 * sh + min(t * ZR2, 2 * SHARE - ZR2), ZR2)],
                sem2)
            for t in range((2 * SHARE + ZR2 - 1) // ZR2)
        ]
        zcps.append(pltpu.async_copy(zcnt_v, cnt_s.at[pl.ds(sh, SHARE)],
                                     sem2))
        pltpu.sync_copy(fsinit_hbm, fsrc_v)
        pltpu.sync_copy(fdinit_hbm, fdst_v)

        def scan_step(g, pos):
            sv = esrc_v[pl.ds(g * 16, 16)]
            dv = edst_v[pl.ds(g * 16, 16)]
            m = jnp.logical_and(sv >= lo, sv < lo + CHUNK)
            csum = plsc.cumsum(jnp.where(m, 1, 0))
            tot = plsc.all_reduce_population_count(m)
            tgt = pos + csum - 1
            row = lax.shift_right_logical(tgt, 7)
            col = jnp.bitwise_and(tgt, 127)
            plsc.store_scatter(fsrc_v, [row, col], sv - lo, mask=m)
            plsc.store_scatter(fdst_v, [row, col], dv, mask=m)
            return pos + tot

        pos = lax.fori_loop(0, NGRP, scan_step,
                            jnp.zeros((16,), jnp.int32), unroll=False)
        nch = (jnp.max(pos) + 127) // 128
        for zc in zcps:
            zc.wait()
        plsc.subcore_barrier()

        def chunk_step(j, carry):
            for half in range(2):
                for k in range(4):
                    kk = half * 4 + k
                    d = fdst_v[j, pl.ds(kk * 16, 16)]
                    a = fsrc_v[j, pl.ds(kk * 16, 16)]
                    fge_v[pl.ds(k * 16, 16)] = 2 * d
                    fgo_v[pl.ds(k * 16, 16)] = 2 * d + 1
                    fae_v[pl.ds(k * 16, 16)] = 2 * a
                    fao_v[pl.ds(k * 16, 16)] = 2 * a + 1
                    fcn_v[pl.ds(k * 16, 16)] = a
                cpe = pltpu.async_copy(mails_hbm.at[fge_v], rowse_v, sem)
                cpo = pltpu.async_copy(mails_hbm.at[fgo_v], rowso_v, sem2)
                cpe.wait()
                pltpu.sync_copy(rowse_v, acc_s.at[fae_v], add=True)
                cpo.wait()
                pltpu.sync_copy(rowso_v, acc_s.at[fao_v], add=True)
                pltpu.sync_copy(ones_v, cnt_s.at[fcn_v], add=True)
            return carry

        lax.fori_loop(0, nch, chunk_step, 0, unroll=False)
        plsc.subcore_barrier()

        nwr = (SHARE * 2 + WR - 1) // WR
        offs = [min(t * WR, 2 * SHARE - WR) for t in range(nwr)]
        bufs = (rowse_v, rowso_v)
        rcp = pltpu.async_copy(acc_s.at[pl.ds(2 * sh + offs[0], WR)],
                               rowse_v.at[pl.ds(0, WR)], sem)
        for t in range(nwr):
            rcp.wait()
            if t + 1 < nwr:
                rcp = pltpu.async_copy(
                    acc_s.at[pl.ds(2 * sh + offs[t + 1], WR)],
                    bufs[(t + 1) % 2].at[pl.ds(0, WR)], sem)
            pltpu.sync_copy(bufs[t % 2].at[pl.ds(0, WR)],
                            msum_hbm.at[pl.ds(2 * (lo + sh) + offs[t], WR)])
        pltpu.sync_copy(cnt_s.at[pl.ds(sh, SHARE)], cbuf_v)
        pltpu.sync_copy(cbuf_v, cnt_hbm.at[pl.ds(lo + sh, SHARE)])
        plsc.subcore_barrier()

    for p in range(NPASS):
        run_pass(p, acc_s, cnt_s)


def _scatter_mean_sc(mails, esrc_p, edst_p):
    fsinit = (CHUNK + jnp.arange(NCH * 128, dtype=jnp.int32) % TRASH
              ).reshape(NCH, 128)
    fdinit = jnp.arange(NCH * 128, dtype=jnp.int32).reshape(NCH, 128)
    zero2d = jnp.zeros((ZR2, 128), jnp.float32)
    zcnt1d = jnp.zeros((SHARE,), jnp.float32)
    ones1d = jnp.ones((64,), jnp.float32)
    mails2 = mails.reshape(-1, 128)

    mesh = plsc.VectorSubcoreMesh(core_axis_name="c", subcore_axis_name="s")
    f = pl.kernel(
        _scatter_body,
        mesh=mesh,
        compiler_params=pltpu.CompilerParams(needs_layout_passes=False),
        out_type=[
            jax.ShapeDtypeStruct((100000, 128), jnp.float32),
            jax.ShapeDtypeStruct((50000,), jnp.float32),
        ],
        scratch_types=[
            pltpu.VMEM((EPS,), jnp.int32),
            pltpu.VMEM((EPS,), jnp.int32),
            pltpu.VMEM((NCH, 128), jnp.int32),
            pltpu.VMEM((NCH, 128), jnp.int32),
            pltpu.VMEM((64,), jnp.int32),
            pltpu.VMEM((64,), jnp.int32),
            pltpu.VMEM((64,), jnp.int32),
            pltpu.VMEM((64,), jnp.int32),
            pltpu.VMEM((64,), jnp.int32),
            pltpu.VMEM((ZR2, 128), jnp.float32),
            pltpu.VMEM((SHARE,), jnp.float32),
            pltpu.VMEM((64,), jnp.float32),
            pltpu.VMEM((64, 128), jnp.float32),
            pltpu.VMEM((64, 128), jnp.float32),
            pltpu.VMEM((SHARE,), jnp.float32),
            pltpu.VMEM_SHARED(((CHUNK + TRASH) * 2, 128), jnp.float32),
            pltpu.VMEM_SHARED((CHUNK + TRASH,), jnp.float32),
            pltpu.SemaphoreType.DMA,
            pltpu.SemaphoreType.DMA,
        ],
    )
    msum2, cnt = f(mails2, esrc_p, edst_p, fsinit, fdinit, zero2d, zcnt1d,
                   ones1d)
    return msum2.reshape(50000, DMSG), cnt


SLOTS = 6144
SPW = SLOTS // (NSC * NSUB)
SCK = 96


def _memscatter_body(out_hbm, nodes_hbm, data_hbm, newmem_ref,
                     nsl_v, dsl_v, gidx_v, sidx_v, grows_v, sem):
    c = lax.axis_index("c")
    s = lax.axis_index("s")
    w = s * NSC + c
    base = pl.multiple_of(w * SPW, 8)

    pltpu.sync_copy(nodes_hbm.at[pl.ds(base, SPW)], nsl_v)
    pltpu.sync_copy(data_hbm.at[pl.ds(base, SPW)], dsl_v)
    for k in range(SPW // SCK):
        for t in range(SCK // 16):
            gidx_v[pl.ds(t * 16, 16)] = dsl_v[pl.ds(k * SCK + t * 16, 16)]
            sidx_v[pl.ds(t * 16, 16)] = nsl_v[pl.ds(k * SCK + t * 16, 16)]
        pltpu.async_copy(out_hbm.at[gidx_v], grows_v, sem).wait()
        pltpu.sync_copy(grows_v, newmem_ref.at[sidx_v])


def _copy_body(in_ref, out_ref):
    out_ref[...] = in_ref[...]


def _mem_update_sc(mem, out9k, nodes6):
    size2 = nodes6.shape[0]
    pos = jnp.arange(size2, dtype=jnp.int32)
    wp = jnp.zeros((mem.shape[0],), jnp.int32).at[nodes6].max(pos)
    dataidx = wp[nodes6]
    nodes_s = jnp.concatenate(
        [nodes6, jnp.full((SLOTS - size2,), nodes6[0], jnp.int32)])
    data_s = jnp.concatenate(
        [dataidx, jnp.full((SLOTS - size2,), dataidx[0], jnp.int32)])

    blk = 2000
    cp = pl.pallas_call(
        _copy_body,
        grid=(mem.shape[0] // blk,),
        in_specs=[pl.BlockSpec((blk, DM), lambda i: (i, 0))],
        out_specs=pl.BlockSpec((blk, DM), lambda i: (i, 0)),
        out_shape=jax.ShapeDtypeStruct(mem.shape, jnp.float32),
    )(mem)

    mesh = plsc.VectorSubcoreMesh(core_axis_name="c", subcore_axis_name="s")
    f = pl.kernel(
        _memscatter_body,
        mesh=mesh,
        compiler_params=pltpu.CompilerParams(needs_layout_passes=False),
        out_type=(),
        scratch_types=[
            pltpu.VMEM((SPW,), jnp.int32),
            pltpu.VMEM((SPW,), jnp.int32),
            pltpu.VMEM((SCK,), jnp.int32),
            pltpu.VMEM((SCK,), jnp.int32),
            pltpu.VMEM((SCK, DM), jnp.float32),
            pltpu.SemaphoreType.DMA,
        ],
    )
    ref = jax.new_ref(cp)
    f(out9k, nodes_s, data_s, ref)
    return ref[...]


GB = 9216
GPW = GB // (NSC * NSUB)
GMC = 16


def _gather_body(mem_hbm, mail_hbm, nodes_hbm, memg_hbm, mailg_hbm,
                 idx_v, memrows_v, mbuf0_v, mbuf1_v, sem0, sem1, sem2):
    c = lax.axis_index("c")
    s = lax.axis_index("s")
    wid = s * NSC + c
    base = pl.multiple_of(wid * GPW, 8)

    pltpu.sync_copy(nodes_hbm.at[pl.ds(base, GPW)], idx_v)
    cpms = [
        pltpu.async_copy(mem_hbm.at[idx_v.at[pl.ds(k * 96, 96)]],
                         memrows_v.at[pl.ds(k * 96, 96)], sem2)
        for k in range(GPW // 96)
    ]

    nmc = GPW // GMC
    bufs = (mbuf0_v, mbuf1_v)
    sems = (sem0, sem1)
    cps = [None, None]
    for t in range(nmc + 1):
        if t < nmc:
            cps[t % 2] = pltpu.async_copy(
                mail_hbm.at[idx_v.at[pl.ds(t * GMC, GMC)]],
                bufs[t % 2], sems[t % 2])
        if t > 0:
            cps[(t - 1) % 2].wait()
            pltpu.sync_copy(bufs[(t - 1) % 2],
                            mailg_hbm.at[pl.ds(base + (t - 1) * GMC, GMC)])

    for cpm in cpms:
        cpm.wait()
    pltpu.sync_copy(memrows_v, memg_hbm.at[pl.ds(base, GPW)])


def _gather_sc(mem, mail2d, nodes_p):
    mesh = plsc.VectorSubcoreMesh(core_axis_name="c", subcore_axis_name="s")
    f = pl.kernel(
        _gather_body,
        mesh=mesh,
        compiler_params=pltpu.CompilerParams(needs_layout_passes=False),
        out_type=[
            jax.ShapeDtypeStruct((GB, DM), jnp.float32),
            jax.ShapeDtypeStruct((GB, MS, DMSG), jnp.float32),
        ],
        scratch_types=[
            pltpu.VMEM((GPW,), jnp.int32),
            pltpu.VMEM((GPW, DM), jnp.float32),
            pltpu.VMEM((GMC, MS, DMSG), jnp.float32),
            pltpu.VMEM((GMC, MS, DMSG), jnp.float32),
            pltpu.SemaphoreType.DMA,
            pltpu.SemaphoreType.DMA,
            pltpu.SemaphoreType.DMA,
        ],
    )
    return f(mem, mail2d, nodes_p)


def _div_body(ms_ref, cnt_ref, out_ref):
    out_ref[...] = ms_ref[...] / jnp.maximum(cnt_ref[...], 1.0)


def _divide_stage(mail_sum, cnt):
    blk = 400
    return pl.pallas_call(
        _div_body,
        grid=(50000 // blk,),
        in_specs=[
            pl.BlockSpec((blk, DMSG), lambda i: (i, 0)),
            pl.BlockSpec((blk, 1), lambda i: (i, 0)),
        ],
        out_specs=pl.BlockSpec((blk, DMSG), lambda i: (i, 0)),
        out_shape=jax.ShapeDtypeStruct((50000, DMSG), jnp.float32),
    )(mail_sum, cnt[:, None])


def kernel(mem, mailbox_mail, mailbox_time, nodes, times, blk_src, blk_dst,
           w_q, b_q, w_k, b_k, w_v, b_v, w_mlp, b_mlp, ln_g, ln_b,
           time_w, time_b, w_src, b_src, w_dst, b_dst, w_out, b_out):
    n = mem.shape[0]
    total = nodes.shape[0]
    size = total // 3
    total_pad = ((total + BLK - 1) // BLK) * BLK

    nodes_p = jnp.concatenate(
        [nodes, jnp.zeros((total_pad - total,), jnp.int32)])
    times_p = jnp.concatenate(
        [times, jnp.zeros((total_pad - total,), jnp.float32)])

    mem_g, mailg = _gather_sc(mem, mailbox_mail, nodes_p)
    mail2 = mailg.reshape(total_pad * MS, DMSG)
    mt = mailbox_time[nodes_p]
    dt2 = (times_p[:, None] - mt).reshape(total_pad * MS, 1)

    out = _attn_stage(
        mem_g, mail2, dt2,
        w_q.T, b_q[None, :],
        w_k[:, :DMSG].T, w_k[:, DMSG:].T, b_k[None, :],
        w_v[:, :DMSG].T, w_v[:, DMSG:].T, b_v[None, :],
        w_mlp.T, b_mlp[None, :], ln_g[None, :], ln_b[None, :],
        time_w[None, :], time_b[None, :], total_pad)

    scores = _edge_stage(out, w_src.T, b_src[None, :], w_dst.T,
                         b_dst[None, :], w_out.T, b_out[None, :], size)

    upd = out[:2 * size]
    new_mem = _mem_update_sc(mem, out, nodes[:2 * size])

    m = upd
    src_mail = jnp.concatenate([m[:size], m[size:]], axis=1)
    dst_mail = jnp.concatenate([m[size:], m[:size]], axis=1)
    mails = jnp.concatenate([src_mail, dst_mail], axis=0)

    es = blk_src.shape[0]
    esrc_p = jnp.concatenate(
        [blk_src, jnp.full((EPAD - es,), n, jnp.int32)])
    edst_p = jnp.concatenate(
        [blk_dst, jnp.zeros((EPAD - es,), jnp.int32)])
    mail_sum, cnt = _scatter_mean_sc(mails, esrc_p, edst_p)
    new_mail = _divide_stage(mail_sum, cnt)
    return scores, new_mem, new_mail


# speedup vs baseline: 1.5290x; 1.0019x over previous
import jax
import jax.numpy as jnp
from jax import lax
from jax.experimental import pallas as pl
from jax.experimental.pallas import tpu as pltpu
from jax.experimental.pallas import tpu_sc as plsc

DM = 128
MS = 8
DMSG = 256
DT = 32
NH = 2
HD = DM // NH

BLK = 512


def _attn_body(mem_ref, mail_ref, dt_ref, wq_ref, bq_ref, wkm_ref, wkt_ref,
               bk_ref, wvm_ref, wvt_ref, bv_ref, wmlp_ref, bmlp_ref,
               lng_ref, lnb_ref, tw_ref, tb_ref, out_ref):
    mem_blk = mem_ref[...]
    mail = mail_ref[...]
    dt = dt_ref[...]

    tf = jnp.cos(dt * tw_ref[...] + tb_ref[...])

    q = mem_blk @ wq_ref[...] + bq_ref[...]
    k2 = mail @ wkm_ref[...] + tf @ wkt_ref[...] + bk_ref[...]
    v2 = mail @ wvm_ref[...] + tf @ wvt_ref[...] + bv_ref[...]

    k3 = k2.reshape(BLK, MS, DM)
    v3 = v2.reshape(BLK, MS, DM)
    q3 = q.reshape(BLK, 1, DM)

    prod = q3 * k3
    lane = lax.broadcasted_iota(jnp.int32, (BLK, MS, DM), 2)
    head0 = lane < HD
    s0 = jnp.sum(jnp.where(head0, prod, 0.0), axis=2)
    s1 = jnp.sum(jnp.where(head0, 0.0, prod), axis=2)

    def _softmax(s):
        s = jnp.where(s >= 0, s, 0.2 * s)
        s = s - jnp.max(s, axis=1, keepdims=True)
        e = jnp.exp(s)
        return e / jnp.sum(e, axis=1, keepdims=True)

    a0 = _softmax(s0)
    a1 = _softmax(s1)
    w3 = jnp.where(head0, a0[:, :, None], a1[:, :, None])
    out = jnp.sum(v3 * w3, axis=1)

    out = out + mem_blk
    mu = jnp.mean(out, axis=1, keepdims=True)
    var = jnp.mean((out - mu) ** 2, axis=1, keepdims=True)
    out = (out - mu) * lax.rsqrt(var + 1e-5) * lng_ref[...] + lnb_ref[...]
    out = jnp.maximum(out @ wmlp_ref[...] + bmlp_ref[...], 0.0)
    out_ref[...] = out


def _attn_stage(mem_g, mail2, dt2, wq_t, b_q, wkm_t, wkt_t, b_k, wvm_t, wvt_t,
                b_v, wmlp_t, b_mlp, ln_g, ln_b, t_w, t_b, total_pad):
    grid = total_pad // BLK
    row_spec = pl.BlockSpec((BLK, DM), lambda i: (i, 0))
    mail_spec = pl.BlockSpec((BLK * MS, DMSG), lambda i: (i, 0))
    dt_spec = pl.BlockSpec((BLK * MS, 1), lambda i: (i, 0))

    def w_spec(shape):
        return pl.BlockSpec(shape, lambda i: (0, 0))

    return pl.pallas_call(
        _attn_body,
        grid=(grid,),
        in_specs=[
            row_spec, mail_spec, dt_spec,
            w_spec((DM, DM)), w_spec((1, DM)),
            w_spec((DMSG, DM)), w_spec((DT, DM)), w_spec((1, DM)),
            w_spec((DMSG, DM)), w_spec((DT, DM)), w_spec((1, DM)),
            w_spec((DM, DM)), w_spec((1, DM)),
            w_spec((1, DM)), w_spec((1, DM)),
            w_spec((1, DT)), w_spec((1, DT)),
        ],
        out_specs=row_spec,
        out_shape=jax.ShapeDtypeStruct((total_pad, DM), jnp.float32),
    )(mem_g, mail2, dt2, wq_t, b_q, wkm_t, wkt_t, b_k, wvm_t, wvt_t, b_v,
      wmlp_t, b_mlp, ln_g, ln_b, t_w, t_b)


def _edge_body(src_ref, dst_ref, neg_ref, ws_ref, bs_ref, wd_ref, bd_ref,
               wo_ref, bo_ref, out_ref):
    hs = src_ref[...] @ ws_ref[...] + bs_ref[...]
    hd = dst_ref[...] @ wd_ref[...] + bd_ref[...]
    hn = neg_ref[...] @ wd_ref[...] + bd_ref[...]
    rp = jnp.maximum(hs + hd, 0.0)
    rn = jnp.maximum(hs + hn, 0.0)
    sp = rp @ wo_ref[...]
    sn = rn @ wo_ref[...]
    out_ref[...] = jnp.concatenate([sp, sn], axis=1) + bo_ref[...]


def _edge_stage(out9k, ws_t, b_src, wd_t, b_dst, wo_t, b_out, size):
    eblk = 600
    grid = size // eblk
    nsb = size // eblk

    return pl.pallas_call(
        _edge_body,
        grid=(grid,),
        in_specs=[
            pl.BlockSpec((eblk, DM), lambda i: (i, 0)),
            pl.BlockSpec((eblk, DM), lambda i: (i + nsb, 0)),
            pl.BlockSpec((eblk, DM), lambda i: (i + 2 * nsb, 0)),
            pl.BlockSpec((DM, DM), lambda i: (0, 0)),
            pl.BlockSpec((1, DM), lambda i: (0, 0)),
            pl.BlockSpec((DM, DM), lambda i: (0, 0)),
            pl.BlockSpec((1, DM), lambda i: (0, 0)),
            pl.BlockSpec((DM, 1), lambda i: (0, 0)),
            pl.BlockSpec((1, 1), lambda i: (0, 0)),
        ],
        out_specs=pl.BlockSpec((eblk, 2), lambda i: (i, 0)),
        out_shape=jax.ShapeDtypeStruct((size, 2), jnp.float32),
    )(out9k, out9k, out9k, ws_t, b_src, wd_t, b_dst, wo_t, b_out)


NSC = 2
NSUB = 16
EPAD = 60160
EPS = EPAD // NSUB
NGRP = EPS // 16
CHUNK = 5248
SHARE = CHUNK // NSUB
TRASH = 128
NCH = 32
ZR2 = 48
WR = 64
NPASS = 5


def _scatter_body(mails_hbm, esrc_hbm, edst_hbm, fsinit_hbm, fdinit_hbm,
                  zero_hbm, zcnt_hbm, ones_hbm, msum_hbm, cnt_hbm,
                  esrc_v, edst_v, fsrc_v, fdst_v, fae_v, fao_v, fge_v, fgo_v,
                  fcn_v, zero_v, zcnt_v, ones_v, rowse_v, rowso_v, cbuf_v,
                  acc_s, cnt_s, sem, sem2):
    c = lax.axis_index("c")
    s = lax.axis_index("s")

    ebase = pl.multiple_of(s * EPS, 16)
    pltpu.sync_copy(esrc_hbm.at[pl.ds(ebase, EPS)], esrc_v)
    pltpu.sync_copy(edst_hbm.at[pl.ds(ebase, EPS)], edst_v)
    pltpu.sync_copy(zero_hbm, zero_v)
    pltpu.sync_copy(zcnt_hbm, zcnt_v)
    pltpu.sync_copy(ones_hbm, ones_v)

    def run_pass(p, acc_s, cnt_s):
        lo = jnp.minimum((NSC * p + c) * CHUNK, 50000 - CHUNK)

        sh = pl.multiple_of(s * SHARE, 8)
        zcps = [
            pltpu.async_copy(
                zero_v,
                acc_s.at[pl.ds(2 * sh + min(t * ZR2, 2 * SHARE - ZR2), ZR2)],
                sem2)
            for t in range((2 * SHARE + ZR2 - 1) // ZR2)
        ]
        zcps.append(pltpu.async_copy(zcnt_v, cnt_s.at[pl.ds(sh, SHARE)],
                                     sem2))
        pltpu.sync_copy(fsinit_hbm, fsrc_v)
        pltpu.sync_copy(fdinit_hbm, fdst_v)

        def scan_step(g, pos):
            sv = esrc_v[pl.ds(g * 16, 16)]
            dv = edst_v[pl.ds(g * 16, 16)]
            m = jnp.logical_and(sv >= lo, sv < lo + CHUNK)
            csum = plsc.cumsum(jnp.where(m, 1, 0))
            tot = plsc.all_reduce_population_count(m)
            tgt = pos + csum - 1
            row = lax.shift_right_logical(tgt, 7)
            col = jnp.bitwise_and(tgt, 127)
            plsc.store_scatter(fsrc_v, [row, col], sv - lo, mask=m)
            plsc.store_scatter(fdst_v, [row, col], dv, mask=m)
            return pos + tot

        pos = lax.fori_loop(0, NGRP, scan_step,
                            jnp.zeros((16,), jnp.int32), unroll=False)
        nch = (jnp.max(pos) + 127) // 128
        for zc in zcps:
            zc.wait()
        plsc.subcore_barrier()

        def chunk_step(j, carry):
            for half in range(2):
                for k in range(4):
                    kk = half * 4 + k
                    d = fdst_v[j, pl.ds(kk * 16, 16)]
                    a = fsrc_v[j, pl.ds(kk * 16, 16)]
                    fge_v[pl.ds(k * 16, 16)] = 2 * d
                    fgo_v[pl.ds(k * 16, 16)] = 2 * d + 1
                    fae_v[pl.ds(k * 16, 16)] = 2 * a
                    fao_v[pl.ds(k * 16, 16)] = 2 * a + 1
                    fcn_v[pl.ds(k * 16, 16)] = a
                cpe = pltpu.async_copy(mails_hbm.at[fge_v], rowse_v, sem)
                cpo = pltpu.async_copy(mails_hbm.at[fgo_v], rowso_v, sem2)
                cpe.wait()
                pltpu.sync_copy(rowse_v, acc_s.at[fae_v], add=True)
                cpo.wait()
                pltpu.sync_copy(rowso_v, acc_s.at[fao_v], add=True)
                pltpu.sync_copy(ones_v, cnt_s.at[fcn_v], add=True)
            return carry

        lax.fori_loop(0, nch, chunk_step, 0, unroll=False)
        plsc.subcore_barrier()

        nwr = (SHARE * 2 + WR - 1) // WR
        offs = [min(t * WR, 2 * SHARE - WR) for t in range(nwr)]
        bufs = (rowse_v, rowso_v)
        rcp = pltpu.async_copy(acc_s.at[pl.ds(2 * sh + offs[0], WR)],
                               rowse_v.at[pl.ds(0, WR)], sem)
        for t in range(nwr):
            rcp.wait()
            if t + 1 < nwr:
                rcp = pltpu.async_copy(
                    acc_s.at[pl.ds(2 * sh + offs[t + 1], WR)],
                    bufs[(t + 1) % 2].at[pl.ds(0, WR)], sem)
            pltpu.sync_copy(bufs[t % 2].at[pl.ds(0, WR)],
                            msum_hbm.at[pl.ds(2 * (lo + sh) + offs[t], WR)])
        pltpu.sync_copy(cnt_s.at[pl.ds(sh, SHARE)], cbuf_v)
        pltpu.sync_copy(cbuf_v, cnt_hbm.at[pl.ds(lo + sh, SHARE)])
        plsc.subcore_barrier()

    for p in range(NPASS):
        run_pass(p, acc_s, cnt_s)


def _scatter_mean_sc(mails, esrc_p, edst_p):
    fsinit = (CHUNK + jnp.arange(NCH * 128, dtype=jnp.int32) % TRASH
              ).reshape(NCH, 128)
    fdinit = jnp.arange(NCH * 128, dtype=jnp.int32).reshape(NCH, 128)
    zero2d = jnp.zeros((ZR2, 128), jnp.float32)
    zcnt1d = jnp.zeros((SHARE,), jnp.float32)
    ones1d = jnp.ones((64,), jnp.float32)
    mails2 = mails.reshape(-1, 128)

    mesh = plsc.VectorSubcoreMesh(core_axis_name="c", subcore_axis_name="s")
    f = pl.kernel(
        _scatter_body,
        mesh=mesh,
        compiler_params=pltpu.CompilerParams(needs_layout_passes=False),
        out_type=[
            jax.ShapeDtypeStruct((100000, 128), jnp.float32),
            jax.ShapeDtypeStruct((50000,), jnp.float32),
        ],
        scratch_types=[
            pltpu.VMEM((EPS,), jnp.int32),
            pltpu.VMEM((EPS,), jnp.int32),
            pltpu.VMEM((NCH, 128), jnp.int32),
            pltpu.VMEM((NCH, 128), jnp.int32),
            pltpu.VMEM((64,), jnp.int32),
            pltpu.VMEM((64,), jnp.int32),
            pltpu.VMEM((64,), jnp.int32),
            pltpu.VMEM((64,), jnp.int32),
            pltpu.VMEM((64,), jnp.int32),
            pltpu.VMEM((ZR2, 128), jnp.float32),
            pltpu.VMEM((SHARE,), jnp.float32),
            pltpu.VMEM((64,), jnp.float32),
            pltpu.VMEM((64, 128), jnp.float32),
            pltpu.VMEM((64, 128), jnp.float32),
            pltpu.VMEM((SHARE,), jnp.float32),
            pltpu.VMEM_SHARED(((CHUNK + TRASH) * 2, 128), jnp.float32),
            pltpu.VMEM_SHARED((CHUNK + TRASH,), jnp.float32),
            pltpu.SemaphoreType.DMA,
            pltpu.SemaphoreType.DMA,
        ],
    )
    msum2, cnt = f(mails2, esrc_p, edst_p, fsinit, fdinit, zero2d, zcnt1d,
                   ones1d)
    return msum2.reshape(50000, DMSG), cnt


SLOTS = 6144
SPW = SLOTS // (NSC * NSUB)
SCK = 96


def _memscatter_body(out_hbm, nodes_hbm, data_hbm, newmem_ref,
                     nsl_v, dsl_v, gidx_v, sidx_v, grows_v, sem):
    c = lax.axis_index("c")
    s = lax.axis_index("s")
    w = s * NSC + c
    base = pl.multiple_of(w * SPW, 8)

    pltpu.sync_copy(nodes_hbm.at[pl.ds(base, SPW)], nsl_v)
    pltpu.sync_copy(data_hbm.at[pl.ds(base, SPW)], dsl_v)
    for k in range(SPW // SCK):
        for t in range(SCK // 16):
            gidx_v[pl.ds(t * 16, 16)] = dsl_v[pl.ds(k * SCK + t * 16, 16)]
            sidx_v[pl.ds(t * 16, 16)] = nsl_v[pl.ds(k * SCK + t * 16, 16)]
        pltpu.async_copy(out_hbm.at[gidx_v], grows_v, sem).wait()
        pltpu.sync_copy(grows_v, newmem_ref.at[sidx_v])


def _copy_body(in_ref, out_ref):
    out_ref[...] = in_ref[...]


def _mem_update_sc(mem, out9k, nodes6):
    size2 = nodes6.shape[0]
    pos = jnp.arange(size2, dtype=jnp.int32)
    wp = jnp.zeros((mem.shape[0],), jnp.int32).at[nodes6].max(pos)
    dataidx = wp[nodes6]
    nodes_s = jnp.concatenate(
        [nodes6, jnp.full((SLOTS - size2,), nodes6[0], jnp.int32)])
    data_s = jnp.concatenate(
        [dataidx, jnp.full((SLOTS - size2,), dataidx[0], jnp.int32)])

    blk = 2000
    cp = pl.pallas_call(
        _copy_body,
        grid=(mem.shape[0] // blk,),
        in_specs=[pl.BlockSpec((blk, DM), lambda i: (i, 0))],
        out_specs=pl.BlockSpec((blk, DM), lambda i: (i, 0)),
        out_shape=jax.ShapeDtypeStruct(mem.shape, jnp.float32),
    )(mem)

    mesh = plsc.VectorSubcoreMesh(core_axis_name="c", subcore_axis_name="s")
    f = pl.kernel(
        _memscatter_body,
        mesh=mesh,
        compiler_params=pltpu.CompilerParams(needs_layout_passes=False),
        out_type=(),
        scratch_types=[
            pltpu.VMEM((SPW,), jnp.int32),
            pltpu.VMEM((SPW,), jnp.int32),
            pltpu.VMEM((SCK,), jnp.int32),
            pltpu.VMEM((SCK,), jnp.int32),
            pltpu.VMEM((SCK, DM), jnp.float32),
            pltpu.SemaphoreType.DMA,
        ],
    )
    ref = jax.new_ref(cp)
    f(out9k, nodes_s, data_s, ref)
    return ref[...]


GB = 9216
GPW = GB // (NSC * NSUB)
GMC = 16


def _gather_body(mem_hbm, mail_hbm, nodes_hbm, memg_hbm, mailg_hbm,
                 idx_v, memrows_v, mbuf0_v, mbuf1_v, sem0, sem1, sem2):
    c = lax.axis_index("c")
    s = lax.axis_index("s")
    wid = s * NSC + c
    base = pl.multiple_of(wid * GPW, 8)

    pltpu.sync_copy(nodes_hbm.at[pl.ds(base, GPW)], idx_v)
    cpms = [
        pltpu.async_copy(mem_hbm.at[idx_v.at[pl.ds(k * 96, 96)]],
                         memrows_v.at[pl.ds(k * 96, 96)], sem2)
        for k in range(GPW // 96)
    ]

    nmc = GPW // GMC
    bufs = (mbuf0_v, mbuf1_v)
    sems = (sem0, sem1)
    cps = [None, None]
    for t in range(nmc + 1):
        if t < nmc:
            cps[t % 2] = pltpu.async_copy(
                mail_hbm.at[idx_v.at[pl.ds(t * GMC, GMC)]],
                bufs[t % 2], sems[t % 2])
        if t > 0:
            cps[(t - 1) % 2].wait()
            pltpu.sync_copy(bufs[(t - 1) % 2],
                            mailg_hbm.at[pl.ds(base + (t - 1) * GMC, GMC)])

    for cpm in cpms:
        cpm.wait()
    pltpu.sync_copy(memrows_v, memg_hbm.at[pl.ds(base, GPW)])


def _gather_sc(mem, mail2d, nodes_p):
    mesh = plsc.VectorSubcoreMesh(core_axis_name="c", subcore_axis_name="s")
    f = pl.kernel(
        _gather_body,
        mesh=mesh,
        compiler_params=pltpu.CompilerParams(needs_layout_passes=False),
        out_type=[
            jax.ShapeDtypeStruct((GB, DM), jnp.float32),
            jax.ShapeDtypeStruct((GB, MS, DMSG), jnp.float32),
        ],
        scratch_types=[
            pltpu.VMEM((GPW,), jnp.int32),
            pltpu.VMEM((GPW, DM), jnp.float32),
            pltpu.VMEM((GMC, MS, DMSG), jnp.float32),
            pltpu.VMEM((GMC, MS, DMSG), jnp.float32),
            pltpu.SemaphoreType.DMA,
            pltpu.SemaphoreType.DMA,
            pltpu.SemaphoreType.DMA,
        ],
    )
    return f(mem, mail2d, nodes_p)


def _div_body(ms_ref, cnt_ref, out_ref):
    out_ref[...] = ms_ref[...] / jnp.maximum(cnt_ref[...], 1.0)


def _divide_stage(mail_sum, cnt):
    blk = 400
    return pl.pallas_call(
        _div_body,
        grid=(50000 // blk,),
        in_specs=[
            pl.BlockSpec((blk, DMSG), lambda i: (i, 0)),
            pl.BlockSpec((blk, 1), lambda i: (i, 0)),
        ],
        out_specs=pl.BlockSpec((blk, DMSG), lambda i: (i, 0)),
        out_shape=jax.ShapeDtypeStruct((50000, DMSG), jnp.float32),
    )(mail_sum, cnt[:, None])


def kernel(mem, mailbox_mail, mailbox_time, nodes, times, blk_src, blk_dst,
           w_q, b_q, w_k, b_k, w_v, b_v, w_mlp, b_mlp, ln_g, ln_b,
           time_w, time_b, w_src, b_src, w_dst, b_dst, w_out, b_out):
    n = mem.shape[0]
    total = nodes.shape[0]
    size = total // 3
    total_pad = ((total + BLK - 1) // BLK) * BLK

    nodes_p = jnp.concatenate(
        [nodes, jnp.zeros((total_pad - total,), jnp.int32)])
    times_p = jnp.concatenate(
        [times, jnp.zeros((total_pad - total,), jnp.float32)])

    mem_g, mailg = _gather_sc(mem, mailbox_mail, nodes_p)
    mail2 = mailg.reshape(total_pad * MS, DMSG)
    mt = mailbox_time[nodes_p]
    dt2 = (times_p[:, None] - mt).reshape(total_pad * MS, 1)

    out = _attn_stage(
        mem_g, mail2, dt2,
        w_q.T, b_q[None, :],
        w_k[:, :DMSG].T, w_k[:, DMSG:].T, b_k[None, :],
        w_v[:, :DMSG].T, w_v[:, DMSG:].T, b_v[None, :],
        w_mlp.T, b_mlp[None, :], ln_g[None, :], ln_b[None, :],
        time_w[None, :], time_b[None, :], total_pad)

    scores = _edge_stage(out, w_src.T, b_src[None, :], w_dst.T,
                         b_dst[None, :], w_out.T, b_out[None, :], size)

    upd = out[:2 * size]
    new_mem = _mem_update_sc(mem, out, nodes[:2 * size])

    m = upd
    src_mail = jnp.concatenate([m[:size], m[size:]], axis=1)
    dst_mail = jnp.concatenate([m[size:], m[:size]], axis=1)
    mails = jnp.concatenate([src_mail, dst_mail], axis=0)

    es = blk_src.shape[0]
    esrc_p = jnp.concatenate(
        [blk_src, jnp.full((EPAD - es,), n, jnp.int32)])
    edst_p = jnp.concatenate(
        [blk_dst, jnp.zeros((EPAD - es,), jnp.int32)])
    mail_sum, cnt = _scatter_mean_sc(mails, esrc_p, edst_p)
    new_mail = _divide_stage(mail_sum, cnt)
    return scores, new_mem, new_mail
